# Initial kernel scaffold; baseline (speedup 1.0000x reference)
import jax
import jax.numpy as jnp
from jax.experimental import pallas as pl


def kernel(x, edge_weight, h_info_node, lin1_W, lin1_b, W_na, na_alphas, sc_alphas, la_alphas, pool_alphas, edge_index, batch):
    raise NotImplementedError("write your pallas kernel here")


# speedup vs baseline: 1.0137x; 1.0137x over previous
import functools
import jax
import jax.numpy as jnp
from jax.experimental import pallas as pl
from jax.experimental.pallas import tpu as pltpu

N = 10000
H = 128
BLK = 400


def _elu(v):
    return jnp.where(v > 0, v, jnp.exp(jnp.minimum(v, 0.0)) - 1.0)


def _layer_body(h_ref, mean_ref, sum_ref, max_ref, gcn_ref, W_ref, w_ref,
                out_ref):
    h = h_ref[...]
    W = W_ref[...]
    o0 = _elu(jnp.dot(gcn_ref[...], W[0], preferred_element_type=jnp.float32))
    o1 = _elu(jnp.dot(h + mean_ref[...], W[1],
                      preferred_element_type=jnp.float32))
    o2 = _elu(jnp.dot(h + sum_ref[...], W[2],
                      preferred_element_type=jnp.float32))
    o3 = _elu(jnp.dot(max_ref[...], W[3], preferred_element_type=jnp.float32))
    o4 = jnp.dot(h, W[4], preferred_element_type=jnp.float32)
    out_ref[...] = (w_ref[0, 0] * o0 + w_ref[0, 1] * o1 + w_ref[0, 2] * o2
                    + w_ref[0, 3] * o3 + w_ref[0, 4] * o4)


@jax.jit
def _layer(h, mean_agg, sum_agg, max_agg, gcn_agg, W_l, w):
    grid = (N // BLK,)
    row = pl.BlockSpec((BLK, H), lambda i: (i, 0))
    full = pl.BlockSpec((5, H, H), lambda i: (0, 0, 0))
    wspec = pl.BlockSpec((1, 5), lambda i: (0, 0), memory_space=pltpu.SMEM)
    return pl.pallas_call(
        _layer_body,
        grid=grid,
        in_specs=[row, row, row, row, row, full, wspec],
        out_specs=row,
        out_shape=jax.ShapeDtypeStruct((N, H), jnp.float32),
    )(h, mean_agg, sum_agg, max_agg, gcn_agg, W_l, w)


def kernel(x, edge_weight, h_info_node, lin1_W, lin1_b, W_na,
           na_alphas, sc_alphas, la_alphas, pool_alphas, edge_index, batch):
    L = W_na.shape[0]
    G = 128
    na_w = jax.nn.softmax(na_alphas, axis=-1)
    sc_w = jax.nn.softmax(sc_alphas, axis=-1)
    la_w = jax.nn.softmax(la_alphas, axis=-1)
    pool_w = jax.nn.softmax(pool_alphas, axis=-1)
    src = edge_index[0]
    dst = edge_index[1]

    h = x @ lin1_W + lin1_b
    deg = jax.ops.segment_sum(edge_weight, dst, num_segments=N)
    inv_sqrt = jnp.where(deg > 0, 1.0 / jnp.sqrt(jnp.maximum(deg, 1e-12)), 0.0)
    gcn_norm = inv_sqrt[src] * inv_sqrt[dst] * edge_weight
    cnt = jax.ops.segment_sum(jnp.ones_like(edge_weight), dst, num_segments=N)
    inv_cnt = 1.0 / jnp.maximum(cnt, 1.0)

    w0 = na_w[0:1]
    jk = []
    for l in range(L):
        msgs = h[src]
        sum_agg = jax.ops.segment_sum(msgs * edge_weight[:, None], dst,
                                      num_segments=N)
        mean_agg = sum_agg * inv_cnt[:, None]
        max_agg = jax.ops.segment_max(msgs, dst, num_segments=N)
        max_agg = jnp.where(jnp.isfinite(max_agg), max_agg, 0.0)
        gcn_agg = jax.ops.segment_sum(msgs * gcn_norm[:, None], dst,
                                      num_segments=N)
        h = _layer(h, mean_agg, sum_agg, max_agg, gcn_agg, W_na[l], w0)
        if l < L - 1:
            jk.append(sc_w[l, 1] * h)
        else:
            jk.append(h)

    stack = jnp.stack(jk, axis=0)
    merge = (la_w[0, 0] * jnp.sum(stack, axis=0)
             + la_w[0, 1] * jnp.mean(stack, axis=0)
             + la_w[0, 2] * jnp.max(stack, axis=0))
    g_sum = jax.ops.segment_sum(merge, batch, num_segments=G)
    g_cnt = jax.ops.segment_sum(jnp.ones((N,), jnp.float32), batch,
                                num_segments=G)
    g_mean = g_sum / jnp.maximum(g_cnt, 1.0)[:, None]
    g_max = jax.ops.segment_max(merge, batch, num_segments=G)
    g_max = jnp.where(jnp.isfinite(g_max), g_max, 0.0)
    readout = (pool_w[0, 0] * g_mean + pool_w[0, 1] * g_sum
               + pool_w[0, 2] * g_max)
    return readout


# speedup vs baseline: 2.8464x; 2.8079x over previous
import functools
import jax
import jax.numpy as jnp
from jax import lax
from jax.experimental import pallas as pl
from jax.experimental.pallas import tpu as pltpu
from jax.experimental.pallas import tpu_sc as plsc

N = 10000
N2 = 10240
E = 320000
H = 128
G = 128
NW = 32
NLOC = N2 // NW
CAP = 16384
BSCAN = 4000
SUB = 128
ACC_ROWS = 5248
DUMP = 5120
NEG = -3.0e38
BLK = 512

_params_sc = pltpu.CompilerParams(needs_layout_passes=False)


def _widx():
    cid = lax.axis_index("c")
    sid = lax.axis_index("s")
    return cid, sid, cid * 16 + sid


def _sc_mesh():
    return plsc.VectorSubcoreMesh(core_axis_name="c", subcore_axis_name="s")


def _rsqrt16(d):
    y = plsc.bitcast(
        jnp.int32(0x5F3759DF) - (plsc.bitcast(d, jnp.int32) >> 1),
        jnp.float32)
    for _i in range(4):
        y = y * (1.5 - 0.5 * d * y * y)
    return y


def _recip16(x):
    y = _rsqrt16(x)
    return y * y


def _make_kernel_a():
    @functools.partial(
        pl.kernel,
        out_type=[
            jax.ShapeDtypeStruct((NW * CAP,), jnp.int32),
            jax.ShapeDtypeStruct((NW * CAP,), jnp.float32),
            jax.ShapeDtypeStruct((NW * CAP,), jnp.int32),
            jax.ShapeDtypeStruct((NW * 16,), jnp.int32),
            jax.ShapeDtypeStruct((N2,), jnp.float32),
            jax.ShapeDtypeStruct((N2,), jnp.float32),
        ],
        mesh=_sc_mesh(),
        compiler_params=_params_sc,
        scratch_types=[
            pltpu.VMEM((BSCAN,), jnp.int32),
            pltpu.VMEM((BSCAN,), jnp.int32),
            pltpu.VMEM((BSCAN,), jnp.float32),
            pltpu.VMEM((CAP,), jnp.int32),
            pltpu.VMEM((CAP,), jnp.float32),
            pltpu.VMEM((CAP,), jnp.int32),
            pltpu.VMEM((328, 16), jnp.float32),
            pltpu.VMEM((NLOC,), jnp.float32),
            pltpu.VMEM((NLOC,), jnp.float32),
            pltpu.VMEM((16,), jnp.int32),
        ],
    )
    def kern(src_hbm, dst_hbm, ew_hbm,
             lsrc_out, lew_out, ldst_out, cnt_out, isq_out, invc_out,
             sbuf, dbuf, ebuf, ls, le, ld, dacc, isq_st, invc_st, c16):
        cid, sid, wid = _widx()
        lo = wid * NLOC
        iota = lax.iota(jnp.int32, 16)
        zf = jnp.zeros((16,), jnp.float32)
        zi = jnp.zeros((16,), jnp.int32)

        def zrow(r, _):
            dacc[r, :] = zf
            return 0
        lax.fori_loop(0, 328, zrow, 0)

        def blk_body(blk, cursor):
            off = pl.multiple_of(blk * BSCAN, 8)
            pltpu.sync_copy(src_hbm.at[pl.ds(off, BSCAN)], sbuf)
            pltpu.sync_copy(dst_hbm.at[pl.ds(off, BSCAN)], dbuf)
            pltpu.sync_copy(ew_hbm.at[pl.ds(off, BSCAN)], ebuf)

            def vreg_body(v, cur):
                sl = pl.ds(v * 16, 16)
                d16 = dbuf[sl]
                s16 = sbuf[sl]
                e16 = ebuf[sl]
                m = (d16 >= lo) & (d16 < lo + NLOC)
                mi = m.astype(jnp.int32)
                pos = cur + plsc.cumsum(mi) - mi
                pos = jnp.minimum(pos, CAP - 1)
                plsc.store_scatter(ls, [pos], s16, mask=m)
                plsc.store_scatter(le, [pos], e16, mask=m)
                plsc.store_scatter(ld, [pos], d16, mask=m)
                return cur + jnp.max(plsc.all_reduce_population_count(m))
            return lax.fori_loop(0, BSCAN // 16, vreg_body, cursor)
        cursor = lax.fori_loop(0, E // BSCAN, blk_body, jnp.int32(0))
        count = jnp.minimum(cursor, CAP)

        nsub = (count + SUB - 1) // SUB
        base = (count // 16) * 16
        dumpdst = jnp.full((16,), cid * 5120 + DUMP, jnp.int32)
        for g in range(SUB // 16 + 1):
            idx = base + g * 16 + iota
            mm = (idx >= count) & (idx < nsub * SUB)
            idxc = jnp.minimum(idx, CAP - 1)
            plsc.store_scatter(ls, [idxc], zi, mask=mm)
            plsc.store_scatter(le, [idxc], zf, mask=mm)
            plsc.store_scatter(ld, [idxc], dumpdst, mask=mm)

        one16 = (iota == 1).astype(jnp.float32)
        def grp_body(g, _):
            sl = pl.ds(g * 16, 16)
            d16 = ld[sl]
            e16 = le[sl]
            for lane in range(16):
                dl = jnp.minimum(d16[lane] - lo, 320)
                mini = jnp.where(iota == 0, e16[lane], one16)
                dacc[dl, :] = dacc[dl, :] + mini
            return 0
        lax.fori_loop(0, (count + 15) // 16, grp_body, 0)

        zcol = jnp.zeros((16,), jnp.int32)
        ocol = jnp.ones((16,), jnp.int32)
        def nb_body(nb, _):
            r16 = nb * 16 + iota
            deg = plsc.load_gather(dacc, [r16, zcol])
            cnt = plsc.load_gather(dacc, [r16, ocol])
            isq = jnp.where(deg > 0, _rsqrt16(jnp.maximum(deg, 1e-12)), 0.0)
            isq_st[pl.ds(nb * 16, 16)] = isq
            invc_st[pl.ds(nb * 16, 16)] = _recip16(jnp.maximum(cnt, 1.0))
            return 0
        lax.fori_loop(0, NLOC // 16, nb_body, 0)

        pltpu.sync_copy(ls, lsrc_out.at[pl.ds(wid * CAP, CAP)])
        pltpu.sync_copy(le, lew_out.at[pl.ds(wid * CAP, CAP)])
        pltpu.sync_copy(ld, ldst_out.at[pl.ds(wid * CAP, CAP)])
        c16[...] = jnp.full((16,), count, jnp.int32)
        pltpu.sync_copy(c16, cnt_out.at[pl.ds(wid * 16, 16)])
        pltpu.sync_copy(isq_st, isq_out.at[pl.ds(wid * NLOC, NLOC)])
        pltpu.sync_copy(invc_st, invc_out.at[pl.ds(wid * NLOC, NLOC)])

    return kern


def _elu(v):
    return jnp.where(v > 0, v, jnp.exp(jnp.minimum(v, 0.0)) - 1.0)


def _lin1_body(x_ref, W_ref, b_ref, out_ref):
    out_ref[...] = (jnp.dot(x_ref[...], W_ref[...],
                            preferred_element_type=jnp.float32)
                    + b_ref[...][0:1, :])


@jax.jit
def _lin1(x_pad, W, b2d):
    row = pl.BlockSpec((BLK, H), lambda i: (i, 0))
    return pl.pallas_call(
        _lin1_body,
        grid=(N2 // BLK,),
        in_specs=[row, pl.BlockSpec((H, H), lambda i: (0, 0)),
                  pl.BlockSpec((8, H), lambda i: (0, 0))],
        out_specs=row,
        out_shape=jax.ShapeDtypeStruct((N2, H), jnp.float32),
    )(x_pad, W, b2d)


def _layer_body(h_ref, sum_ref, max_ref, gcn_ref, ic_ref, W_ref, w_ref,
                out_ref):
    h = h_ref[...]
    W = W_ref[...]
    s = sum_ref[...]
    o0 = _elu(jnp.dot(gcn_ref[...], W[0], preferred_element_type=jnp.float32))
    o1 = _elu(jnp.dot(h + s * ic_ref[...], W[1],
                      preferred_element_type=jnp.float32))
    o2 = _elu(jnp.dot(h + s, W[2], preferred_element_type=jnp.float32))
    o3 = _elu(jnp.dot(max_ref[...], W[3], preferred_element_type=jnp.float32))
    o4 = jnp.dot(h, W[4], preferred_element_type=jnp.float32)
    out_ref[...] = (w_ref[0, 0] * o0 + w_ref[0, 1] * o1 + w_ref[0, 2] * o2
                    + w_ref[0, 3] * o3 + w_ref[0, 4] * o4)


@jax.jit
def _layer(h, sum_agg, max_agg, gcn_agg, invc2d, W_l, w):
    row = pl.BlockSpec((BLK, H), lambda i: (i, 0))
    return pl.pallas_call(
        _layer_body,
        grid=(N2 // BLK,),
        in_specs=[row, row, row, row, row,
                  pl.BlockSpec((5, H, H), lambda i: (0, 0, 0)),
                  pl.BlockSpec((1, 5), lambda i: (0, 0),
                               memory_space=pltpu.SMEM)],
        out_specs=row,
        out_shape=jax.ShapeDtypeStruct((N2, H), jnp.float32),
    )(h, sum_agg, max_agg, gcn_agg, invc2d, W_l, w)


def _layer3_body(h_ref, sum_ref, max_ref, gcn_ref, ic_ref, W_ref, w_ref,
                 h1_ref, h2_ref, m_ref, out_ref):
    h = h_ref[...]
    W = W_ref[...]
    s = sum_ref[...]
    o0 = _elu(jnp.dot(gcn_ref[...], W[0], preferred_element_type=jnp.float32))
    o1 = _elu(jnp.dot(h + s * ic_ref[...], W[1],
                      preferred_element_type=jnp.float32))
    o2 = _elu(jnp.dot(h + s, W[2], preferred_element_type=jnp.float32))
    o3 = _elu(jnp.dot(max_ref[...], W[3], preferred_element_type=jnp.float32))
    o4 = jnp.dot(h, W[4], preferred_element_type=jnp.float32)
    h3 = (w_ref[0, 0] * o0 + w_ref[0, 1] * o1 + w_ref[0, 2] * o2
          + w_ref[0, 3] * o3 + w_ref[0, 4] * o4)
    j1 = m_ref[0, 0] * h1_ref[...]
    j2 = m_ref[0, 1] * h2_ref[...]
    tot = j1 + j2 + h3
    mx = jnp.maximum(jnp.maximum(j1, j2), h3)
    out_ref[...] = ((m_ref[0, 2] + m_ref[0, 3] / 3.0) * tot
                    + m_ref[0, 4] * mx)


@jax.jit
def _layer3(h, sum_agg, max_agg, gcn_agg, invc2d, W_l, w, h1, h2, msc):
    row = pl.BlockSpec((BLK, H), lambda i: (i, 0))
    return pl.pallas_call(
        _layer3_body,
        grid=(N2 // BLK,),
        in_specs=[row, row, row, row, row,
                  pl.BlockSpec((5, H, H), lambda i: (0, 0, 0)),
                  pl.BlockSpec((1, 5), lambda i: (0, 0),
                               memory_space=pltpu.SMEM),
                  row, row,
                  pl.BlockSpec((1, 8), lambda i: (0, 0),
                               memory_space=pltpu.SMEM)],
        out_specs=row,
        out_shape=jax.ShapeDtypeStruct((N2, H), jnp.float32),
    )(h, sum_agg, max_agg, gcn_agg, invc2d, W_l, w, h1, h2, msc)


def _make_kernel_a2():
    @functools.partial(
        pl.kernel,
        out_type=[jax.ShapeDtypeStruct((NW * CAP,), jnp.float32)],
        mesh=_sc_mesh(),
        compiler_params=_params_sc,
        scratch_types=[
            pltpu.VMEM((N2,), jnp.float32),
            pltpu.VMEM((SUB,), jnp.int32),
            pltpu.VMEM((SUB,), jnp.int32),
            pltpu.VMEM((SUB,), jnp.float32),
            pltpu.VMEM((SUB,), jnp.float32),
            pltpu.VMEM((16,), jnp.int32),
        ],
    )
    def kern(lsrc, ldst, lew, cnts, isq_hbm, lgw_out,
             isqt, srcv, dstv, eww, gww, c16):
        cid, sid, wid = _widx()
        pltpu.sync_copy(isq_hbm, isqt)
        pltpu.sync_copy(cnts.at[pl.ds(wid * 16, 16)], c16)
        count = c16[...][0]
        nsub = (count + SUB - 1) // SUB

        def sub_body(i, _):
            base = pl.multiple_of(wid * CAP + i * SUB, 8)
            pltpu.sync_copy(lsrc.at[pl.ds(base, SUB)], srcv)
            pltpu.sync_copy(ldst.at[pl.ds(base, SUB)], dstv)
            pltpu.sync_copy(lew.at[pl.ds(base, SUB)], eww)

            def grp(g, _2):
                sl16 = pl.ds(g * 16, 16)
                s16 = srcv[sl16]
                d16 = jnp.minimum(dstv[sl16], N2 - 1)
                gww[sl16] = (eww[sl16] * plsc.load_gather(isqt, [s16])
                             * plsc.load_gather(isqt, [d16]))
                return 0
            lax.fori_loop(0, SUB // 16, grp, 0)
            pltpu.sync_copy(gww, lgw_out.at[pl.ds(base, SUB)])
            return 0
        lax.fori_loop(0, nsub, sub_body, 0)

    return kern


SUBD = 32


def _make_kernel_d():
    @functools.partial(
        pl.kernel,
        out_type=[
            jax.ShapeDtypeStruct((N2, H), jnp.float32),
            jax.ShapeDtypeStruct((N2, H), jnp.float32),
            jax.ShapeDtypeStruct((N2, H), jnp.float32),
        ],
        mesh=_sc_mesh(),
        compiler_params=_params_sc,
        scratch_types=[
            pltpu.VMEM((328, H), jnp.float32),
            pltpu.VMEM((328, H), jnp.float32),
            pltpu.VMEM((328, H), jnp.float32),
            pltpu.VMEM((SUBD,), jnp.int32),
            pltpu.VMEM((SUBD,), jnp.int32),
            pltpu.VMEM((SUBD,), jnp.float32),
            pltpu.VMEM((SUBD,), jnp.float32),
            pltpu.VMEM((SUBD, H), jnp.float32),
            pltpu.VMEM((16,), jnp.int32),
            pltpu.SemaphoreType.DMA,
        ],
    )
    def kern(h_hbm, lsrc, lew, lgw, ldst, cnts,
             sum_out, gcn_out, mx_out,
             sacc, gacc, macc, srcv, dstv, eww, gww, rows, c16, sem):
        cid, sid, wid = _widx()
        lo = wid * NLOC
        zf = jnp.zeros((16,), jnp.float32)
        negf = jnp.full((16,), NEG, jnp.float32)

        def zr(r, _):
            for q in range(H // 16):
                sl = pl.ds(q * 16, 16)
                sacc[r, sl] = zf
                gacc[r, sl] = zf
                macc[r, sl] = negf
            return 0
        lax.fori_loop(0, 328, zr, 0)

        pltpu.sync_copy(cnts.at[pl.ds(wid * 16, 16)], c16)
        count = c16[...][0]
        nsub = (count + SUBD - 1) // SUBD

        def sub_body(i, _):
            base = pl.multiple_of(wid * CAP + i * SUBD, 8)
            pltpu.sync_copy(lsrc.at[pl.ds(base, SUBD)], srcv)
            pltpu.sync_copy(ldst.at[pl.ds(base, SUBD)], dstv)
            pltpu.sync_copy(lew.at[pl.ds(base, SUBD)], eww)
            pltpu.sync_copy(lgw.at[pl.ds(base, SUBD)], gww)
            pltpu.async_copy(h_hbm.at[srcv], rows, sem).wait()

            def grp(g, _2):
                sl16 = pl.ds(g * 16, 16)
                d16 = dstv[sl16]
                e16 = eww[sl16]
                q16 = gww[sl16]
                for lane in range(16):
                    e = g * 16 + lane
                    dl = jnp.minimum(d16[lane] - lo, 320)
                    w = e16[lane]
                    qq = q16[lane]
                    for qs in range(H // 16):
                        sl = pl.ds(qs * 16, 16)
                        r = rows[e, sl]
                        sacc[dl, sl] = sacc[dl, sl] + r * w
                        gacc[dl, sl] = gacc[dl, sl] + r * qq
                        macc[dl, sl] = jnp.maximum(macc[dl, sl], r)
                return 0
            lax.fori_loop(0, SUBD // 16, grp, 0)
            return 0
        lax.fori_loop(0, nsub, sub_body, 0)

        def fx(r, _):
            for q in range(H // 16):
                sl = pl.ds(q * 16, 16)
                v = macc[r, sl]
                macc[r, sl] = jnp.where(v < -1.0e38, 0.0, v)
            return 0
        lax.fori_loop(0, NLOC, fx, 0)
        obase = pl.multiple_of(lo, 8)
        pltpu.sync_copy(sacc.at[pl.ds(0, NLOC)], sum_out.at[pl.ds(obase, NLOC)])
        pltpu.sync_copy(gacc.at[pl.ds(0, NLOC)], gcn_out.at[pl.ds(obase, NLOC)])
        pltpu.sync_copy(macc.at[pl.ds(0, NLOC)], mx_out.at[pl.ds(obase, NLOC)])

    return kern


def _make_kernel_p():
    BP = 2000

    @functools.partial(
        pl.kernel,
        out_type=[jax.ShapeDtypeStruct((NW, 4, H), jnp.float32)],
        mesh=_sc_mesh(),
        compiler_params=_params_sc,
        scratch_types=[
            pltpu.VMEM((BP,), jnp.int32),
            pltpu.VMEM((BP + SUB,), jnp.int32),
            pltpu.VMEM((BP + SUB,), jnp.int32),
            pltpu.VMEM((8, H), jnp.float32),
            pltpu.VMEM((8, H), jnp.float32),
            pltpu.VMEM((SUB, H), jnp.float32),
            pltpu.VMEM((16,), jnp.float32),
            pltpu.VMEM((8, H), jnp.float32),
            pltpu.SMEM((8,), jnp.int32),
            pltpu.SemaphoreType.DMA,
        ],
    )
    def kern(merge_hbm, batch_hbm, pw_hbm, ro_out,
             bbuf, nid, gid, psum, pmax, rows, pwv, ost, cnt_s, sem):
        cid, sid, wid = _widx()
        glo = wid * 4
        iota = lax.iota(jnp.int32, 16)
        zf = jnp.zeros((16,), jnp.float32)
        negf = jnp.full((16,), NEG, jnp.float32)
        for r in range(8):
            for q in range(H // 16):
                psum[r, pl.ds(q * 16, 16)] = zf
                pmax[r, pl.ds(q * 16, 16)] = negf
            cnt_s[r] = 0
        pltpu.sync_copy(pw_hbm, pwv)

        for b5 in range(N // BP):
            pltpu.sync_copy(batch_hbm.at[pl.ds(b5 * BP, BP)], bbuf)

            def vb(v, cur):
                sl = pl.ds(v * 16, 16)
                b16 = bbuf[sl]
                m = (b16 >= glo) & (b16 < glo + 4)
                mi = m.astype(jnp.int32)
                pos = cur + plsc.cumsum(mi) - mi
                plsc.store_scatter(nid, [pos], b5 * BP + v * 16 + iota, mask=m)
                plsc.store_scatter(gid, [pos], b16 - glo, mask=m)
                return cur + jnp.max(plsc.all_reduce_population_count(m))
            cursor = lax.fori_loop(0, BP // 16, vb, jnp.int32(0))

            nsub = (cursor + SUB - 1) // SUB
            tbase = (cursor // 16) * 16
            for g in range(SUB // 16 + 1):
                idx = tbase + g * 16 + iota
                mm = (idx >= cursor) & (idx < nsub * SUB)
                plsc.store_scatter(nid, [idx], jnp.zeros((16,), jnp.int32),
                                   mask=mm)
                plsc.store_scatter(gid, [idx], jnp.full((16,), 4, jnp.int32),
                                   mask=mm)

            def sub_body(i, _):
                pltpu.async_copy(merge_hbm.at[nid.at[pl.ds(i * SUB, SUB)]],
                                 rows, sem).wait()

                def grp(g, _2):
                    g16 = gid[pl.ds(i * SUB + g * 16, 16)]
                    for lane in range(16):
                        e = g * 16 + lane
                        gl = jnp.minimum(g16[lane], 4)
                        cnt_s[gl] = cnt_s[gl] + 1
                        for qs in range(H // 16):
                            sl = pl.ds(qs * 16, 16)
                            r = rows[e, sl]
                            psum[gl, sl] = psum[gl, sl] + r
                            pmax[gl, sl] = jnp.maximum(pmax[gl, sl], r)
                    return 0
                lax.fori_loop(0, SUB // 16, grp, 0)
                return 0
            lax.fori_loop(0, nsub, sub_body, 0)

        pw = pwv[...]
        for gg in range(4):
            c = jnp.full((16,), cnt_s[gg], jnp.int32).astype(jnp.float32)
            coef = pw[0] * _recip16(jnp.maximum(c, 1.0)) + pw[1]
            for qs in range(H // 16):
                sl = pl.ds(qs * 16, 16)
                mx = pmax[gg, sl]
                mx = jnp.where(mx < -1.0e38, 0.0, mx)
                ost[gg, sl] = coef * psum[gg, sl] + pw[2] * mx
        pltpu.sync_copy(ost.at[pl.ds(0, 4)], ro_out.at[wid])

    return kern


_KS = None


def _get_kernels():
    global _KS
    if _KS is None:
        _KS = (_make_kernel_a(), _make_kernel_a2(), _make_kernel_d(),
               _make_kernel_p())
    return _KS


def kernel(x, edge_weight, h_info_node, lin1_W, lin1_b, W_na,
           na_alphas, sc_alphas, la_alphas, pool_alphas, edge_index, batch):
    ka, ka2, kd, kp = _get_kernels()
    L = W_na.shape[0]
    na_w = jax.nn.softmax(na_alphas, axis=-1)
    sc_w = jax.nn.softmax(sc_alphas, axis=-1)
    la_w = jax.nn.softmax(la_alphas, axis=-1)
    pool_w = jax.nn.softmax(pool_alphas, axis=-1)
    src = edge_index[0]
    dst = edge_index[1]

    x_pad = jnp.pad(x, ((0, N2 - N), (0, 0)))
    b2d = jnp.broadcast_to(lin1_b[None, :], (8, H))

    lsrc, lew, ldst, cnts, isq, invc = ka(src, dst, edge_weight)
    lgw = ka2(lsrc, ldst, lew, cnts, isq)[0]
    h = _lin1(x_pad, lin1_W, b2d)
    invc2d = jnp.broadcast_to(invc[:, None], (N2, H))

    w0 = na_w[0:1]
    msc = jnp.concatenate([sc_w[0, 1:2], sc_w[1, 1:2], la_w[0],
                           jnp.zeros((3,), jnp.float32)])[None, :]

    hs = []
    for l in range(L):
        sum_agg, gcn_agg, max_agg = kd(h, lsrc, lew, lgw, ldst, cnts)
        if l < L - 1:
            h = _layer(h, sum_agg, max_agg, gcn_agg, invc2d, W_na[l], w0)
            hs.append(h)
        else:
            merge = _layer3(h, sum_agg, max_agg, gcn_agg, invc2d, W_na[l],
                            w0, hs[0], hs[1], msc)

    pw = jnp.concatenate([pool_w[0], jnp.zeros((13,), jnp.float32)])
    ro = kp(merge, batch, pw)[0]
    return ro.reshape(G, H)


# speedup vs baseline: 4.5753x; 1.6074x over previous
import functools
import jax
import jax.numpy as jnp
from jax import lax
from jax.experimental import pallas as pl
from jax.experimental.pallas import tpu as pltpu
from jax.experimental.pallas import tpu_sc as plsc

N = 10000
N2 = 10240
E = 320000
H = 128
G = 128
NW = 32
NLOC = N2 // NW
CAP = 16384
BSCAN = 6400
SUB = 128
CAP2 = CAP // 2
HLOC = 160
ACC_ROWS = 5248
DUMP = 5120
NEG = -3.0e38
BLK = 512

_params_sc = pltpu.CompilerParams(needs_layout_passes=False)


def _widx():
    cid = lax.axis_index("c")
    sid = lax.axis_index("s")
    return cid, sid, cid * 16 + sid


def _sc_mesh():
    return plsc.VectorSubcoreMesh(core_axis_name="c", subcore_axis_name="s")


def _rsqrt16(d):
    y = plsc.bitcast(
        jnp.int32(0x5F3759DF) - (plsc.bitcast(d, jnp.int32) >> 1),
        jnp.float32)
    for _i in range(4):
        y = y * (1.5 - 0.5 * d * y * y)
    return y


def _recip16(x):
    y = _rsqrt16(x)
    return y * y


def _make_kernel_a():
    @functools.partial(
        pl.kernel,
        out_type=[
            jax.ShapeDtypeStruct((NW * CAP,), jnp.int32),
            jax.ShapeDtypeStruct((NW * CAP,), jnp.float32),
            jax.ShapeDtypeStruct((NW * CAP,), jnp.int32),
            jax.ShapeDtypeStruct((NW * 16,), jnp.int32),
            jax.ShapeDtypeStruct((N2,), jnp.float32),
            jax.ShapeDtypeStruct((N2,), jnp.float32),
        ],
        mesh=_sc_mesh(),
        compiler_params=_params_sc,
        scratch_types=[
            pltpu.VMEM((BSCAN,), jnp.int32),
            pltpu.VMEM((BSCAN,), jnp.int32),
            pltpu.VMEM((BSCAN,), jnp.float32),
            pltpu.VMEM((CAP,), jnp.int32),
            pltpu.VMEM((CAP,), jnp.float32),
            pltpu.VMEM((CAP,), jnp.int32),
            pltpu.VMEM((CAP,), jnp.int32),
            pltpu.VMEM((CAP,), jnp.float32),
            pltpu.VMEM((CAP,), jnp.int32),
            pltpu.VMEM((48, H), jnp.float32),
            pltpu.VMEM((NLOC,), jnp.float32),
            pltpu.VMEM((NLOC,), jnp.float32),
            pltpu.VMEM((16,), jnp.int32),
        ],
    )
    def kern(src_hbm, dst_hbm, ew_hbm,
             lsrc_out, lew_out, ldst_out, cnt_out, isq_out, invc_out,
             sbuf, dbuf, ebuf, ls, le, ld, ls2, le2, ld2, dacc, isq_st, invc_st, c16):
        cid, sid, wid = _widx()
        lo = wid * NLOC
        iota = lax.iota(jnp.int32, 16)
        zf = jnp.zeros((16,), jnp.float32)
        zi = jnp.zeros((16,), jnp.int32)

        def zrow(r, _):
            for q in range(H // 16):
                dacc[r, pl.ds(q * 16, 16)] = zf
            return 0
        lax.fori_loop(0, 48, zrow, 0)

        def blk_body(blk, cursor):
            off = pl.multiple_of(blk * BSCAN, 8)
            pltpu.sync_copy(src_hbm.at[pl.ds(off, BSCAN)], sbuf)
            pltpu.sync_copy(dst_hbm.at[pl.ds(off, BSCAN)], dbuf)
            pltpu.sync_copy(ew_hbm.at[pl.ds(off, BSCAN)], ebuf)

            def vreg_body(v, cur):
                sl = pl.ds(v * 16, 16)
                d16 = dbuf[sl]
                s16 = sbuf[sl]
                e16 = ebuf[sl]
                m = (d16 >= lo) & (d16 < lo + NLOC)
                mi = m.astype(jnp.int32)
                pos = cur + plsc.cumsum(mi) - mi
                pos = jnp.minimum(pos, CAP - 1)
                plsc.store_scatter(ls, [pos], s16, mask=m)
                plsc.store_scatter(le, [pos], e16, mask=m)
                plsc.store_scatter(ld, [pos], d16, mask=m)
                return cur + jnp.max(plsc.all_reduce_population_count(m))
            return lax.fori_loop(0, BSCAN // 16, vreg_body, cursor)
        cursor = lax.fori_loop(0, E // BSCAN, blk_body, jnp.int32(0))
        count = jnp.minimum(cursor, CAP)

        mid = lo + HLOC

        def rp_body(v, carry):
            c0, c1 = carry
            sl = pl.ds(v * 16, 16)
            d16 = ld[sl]
            s16 = ls[sl]
            e16 = le[sl]
            valid = (v * 16 + iota) < count
            m0 = valid & (d16 < mid)
            m1 = valid & (d16 >= mid)
            m0i = m0.astype(jnp.int32)
            m1i = m1.astype(jnp.int32)
            pos0 = jnp.minimum(c0 + plsc.cumsum(m0i) - m0i, CAP2 - 1)
            pos1 = jnp.minimum(c1 + plsc.cumsum(m1i) - m1i, CAP2 - 1) + CAP2
            plsc.store_scatter(ls2, [pos0], s16, mask=m0)
            plsc.store_scatter(le2, [pos0], e16, mask=m0)
            plsc.store_scatter(ld2, [pos0], d16, mask=m0)
            plsc.store_scatter(ls2, [pos1], s16, mask=m1)
            plsc.store_scatter(le2, [pos1], e16, mask=m1)
            plsc.store_scatter(ld2, [pos1], d16, mask=m1)
            c0 = c0 + jnp.max(plsc.all_reduce_population_count(m0))
            c1 = c1 + jnp.max(plsc.all_reduce_population_count(m1))
            return c0, c1
        c0, c1 = lax.fori_loop(0, (count + 15) // 16, rp_body,
                               (jnp.int32(0), jnp.int32(0)))
        c0 = jnp.minimum(c0, CAP2)
        c1 = jnp.minimum(c1, CAP2)

        dumpdst = jnp.full((16,), cid * 5120 + DUMP, jnp.int32)
        for half, ch in ((0, c0), (1, c1)):
            nsub_h = (ch + SUB - 1) // SUB
            base_h = (ch // 16) * 16
            for g in range(SUB // 16 + 1):
                idx = base_h + g * 16 + iota
                mm = (idx >= ch) & (idx < nsub_h * SUB)
                idxc = jnp.minimum(idx, CAP2 - 1) + half * CAP2
                plsc.store_scatter(ls2, [idxc], zi, mask=mm)
                plsc.store_scatter(le2, [idxc], zf, mask=mm)
                plsc.store_scatter(ld2, [idxc], dumpdst, mask=mm)

        one16 = (iota == 1).astype(jnp.float32)
        def grp_body(g, _):
            sl = pl.ds(g * 16, 16)
            d16 = ld[sl]
            e16 = le[sl]
            for lane in range(16):
                valid = (g * 16 + lane) < count
                dl = jnp.clip(d16[lane] - lo, 0, 320)
                row = dl >> 3
                col = (dl & 7) * 16
                sl2 = pl.ds(col, 16)
                mini = jnp.where(iota == 0, e16[lane], one16)
                mini = jnp.where(valid, mini, 0.0)
                dacc[row, sl2] = dacc[row, sl2] + mini
            return 0
        lax.fori_loop(0, (count + 15) // 16, grp_body, 0)

        def nb_body(nb, _):
            r16 = nb * 16 + iota
            row16 = r16 >> 3
            col16 = (r16 & 7) * 16
            deg = plsc.load_gather(dacc, [row16, col16])
            cnt = plsc.load_gather(dacc, [row16, col16 + 1])
            isq = jnp.where(deg > 0, _rsqrt16(jnp.maximum(deg, 1e-12)), 0.0)
            isq_st[pl.ds(nb * 16, 16)] = isq
            invc_st[pl.ds(nb * 16, 16)] = _recip16(jnp.maximum(cnt, 1.0))
            return 0
        lax.fori_loop(0, NLOC // 16, nb_body, 0)

        pltpu.sync_copy(ls2, lsrc_out.at[pl.ds(wid * CAP, CAP)])
        pltpu.sync_copy(le2, lew_out.at[pl.ds(wid * CAP, CAP)])
        pltpu.sync_copy(ld2, ldst_out.at[pl.ds(wid * CAP, CAP)])
        iota16 = lax.iota(jnp.int32, 16)
        c16[...] = jnp.where(iota16 == 0, c0, jnp.where(iota16 == 1, c1, 0))
        pltpu.sync_copy(c16, cnt_out.at[pl.ds(wid * 16, 16)])
        pltpu.sync_copy(isq_st, isq_out.at[pl.ds(wid * NLOC, NLOC)])
        pltpu.sync_copy(invc_st, invc_out.at[pl.ds(wid * NLOC, NLOC)])

    return kern


def _elu(v):
    return jnp.where(v > 0, v, jnp.exp(jnp.minimum(v, 0.0)) - 1.0)


def _lin1_body(x_ref, W_ref, b_ref, out_ref):
    out_ref[...] = (jnp.dot(x_ref[...], W_ref[...],
                            preferred_element_type=jnp.float32)
                    + b_ref[...][0:1, :])


@jax.jit
def _lin1(x_pad, W, b2d):
    row = pl.BlockSpec((BLK, H), lambda i: (i, 0))
    return pl.pallas_call(
        _lin1_body,
        grid=(N2 // BLK,),
        in_specs=[row, pl.BlockSpec((H, H), lambda i: (0, 0)),
                  pl.BlockSpec((8, H), lambda i: (0, 0))],
        out_specs=row,
        out_shape=jax.ShapeDtypeStruct((N2, H), jnp.float32),
    )(x_pad, W, b2d)


def _layer_body(h_ref, sum_ref, max_ref, gcn_ref, ic_ref, W_ref, w_ref,
                out_ref):
    h = h_ref[...]
    W = W_ref[...]
    s = sum_ref[...]
    o0 = _elu(jnp.dot(gcn_ref[...], W[0], preferred_element_type=jnp.float32))
    o1 = _elu(jnp.dot(h + s * ic_ref[...], W[1],
                      preferred_element_type=jnp.float32))
    o2 = _elu(jnp.dot(h + s, W[2], preferred_element_type=jnp.float32))
    o3 = _elu(jnp.dot(max_ref[...], W[3], preferred_element_type=jnp.float32))
    o4 = jnp.dot(h, W[4], preferred_element_type=jnp.float32)
    out_ref[...] = (w_ref[0, 0] * o0 + w_ref[0, 1] * o1 + w_ref[0, 2] * o2
                    + w_ref[0, 3] * o3 + w_ref[0, 4] * o4)


@jax.jit
def _layer(h, sum_agg, max_agg, gcn_agg, invc2d, W_l, w):
    row = pl.BlockSpec((BLK, H), lambda i: (i, 0))
    return pl.pallas_call(
        _layer_body,
        grid=(N2 // BLK,),
        in_specs=[row, row, row, row, row,
                  pl.BlockSpec((5, H, H), lambda i: (0, 0, 0)),
                  pl.BlockSpec((1, 5), lambda i: (0, 0),
                               memory_space=pltpu.SMEM)],
        out_specs=row,
        out_shape=jax.ShapeDtypeStruct((N2, H), jnp.float32),
    )(h, sum_agg, max_agg, gcn_agg, invc2d, W_l, w)


def _layer3_body(h_ref, sum_ref, max_ref, gcn_ref, ic_ref, W_ref, w_ref,
                 h1_ref, h2_ref, m_ref, out_ref):
    h = h_ref[...]
    W = W_ref[...]
    s = sum_ref[...]
    o0 = _elu(jnp.dot(gcn_ref[...], W[0], preferred_element_type=jnp.float32))
    o1 = _elu(jnp.dot(h + s * ic_ref[...], W[1],
                      preferred_element_type=jnp.float32))
    o2 = _elu(jnp.dot(h + s, W[2], preferred_element_type=jnp.float32))
    o3 = _elu(jnp.dot(max_ref[...], W[3], preferred_element_type=jnp.float32))
    o4 = jnp.dot(h, W[4], preferred_element_type=jnp.float32)
    h3 = (w_ref[0, 0] * o0 + w_ref[0, 1] * o1 + w_ref[0, 2] * o2
          + w_ref[0, 3] * o3 + w_ref[0, 4] * o4)
    j1 = m_ref[0, 0] * h1_ref[...]
    j2 = m_ref[0, 1] * h2_ref[...]
    tot = j1 + j2 + h3
    mx = jnp.maximum(jnp.maximum(j1, j2), h3)
    out_ref[...] = ((m_ref[0, 2] + m_ref[0, 3] / 3.0) * tot
                    + m_ref[0, 4] * mx)


@jax.jit
def _layer3(h, sum_agg, max_agg, gcn_agg, invc2d, W_l, w, h1, h2, msc):
    row = pl.BlockSpec((BLK, H), lambda i: (i, 0))
    return pl.pallas_call(
        _layer3_body,
        grid=(N2 // BLK,),
        in_specs=[row, row, row, row, row,
                  pl.BlockSpec((5, H, H), lambda i: (0, 0, 0)),
                  pl.BlockSpec((1, 5), lambda i: (0, 0),
                               memory_space=pltpu.SMEM),
                  row, row,
                  pl.BlockSpec((1, 8), lambda i: (0, 0),
                               memory_space=pltpu.SMEM)],
        out_specs=row,
        out_shape=jax.ShapeDtypeStruct((N2, H), jnp.float32),
    )(h, sum_agg, max_agg, gcn_agg, invc2d, W_l, w, h1, h2, msc)


SUBD = 128
LCH = 1024


def _make_kernel_d():
    @functools.partial(
        pl.kernel,
        out_type=[
            jax.ShapeDtypeStruct((N2, H), jnp.float32),
            jax.ShapeDtypeStruct((N2, H), jnp.float32),
            jax.ShapeDtypeStruct((N2, H), jnp.float32),
        ],
        mesh=_sc_mesh(),
        compiler_params=_params_sc,
        scratch_types=[
            pltpu.VMEM((168, H), jnp.float32),
            pltpu.VMEM((168, H), jnp.float32),
            pltpu.VMEM((168, H), jnp.float32),
            pltpu.VMEM((N2,), jnp.float32),
            pltpu.VMEM((LCH,), jnp.int32),
            pltpu.VMEM((LCH,), jnp.int32),
            pltpu.VMEM((LCH,), jnp.float32),
            pltpu.VMEM((SUBD, H), jnp.float32),
            pltpu.VMEM((16,), jnp.int32),
            pltpu.SemaphoreType.DMA,
        ],
    )
    def kern(h_hbm, lsrc, lew, ldst, cnts, isq_hbm,
             sum_out, gcn_out, mx_out,
             sacc, gacc, macc, isqt, lsb, ldb, leb, rows, c16, sem):
        cid, sid, wid = _widx()
        lo = wid * NLOC
        zf = jnp.zeros((16,), jnp.float32)
        negf = jnp.full((16,), NEG, jnp.float32)
        pltpu.sync_copy(isq_hbm, isqt)
        pltpu.sync_copy(cnts.at[pl.ds(wid * 16, 16)], c16)
        cboth = c16[...]

        for half in range(2):
            h_lo = lo + half * HLOC
            count = cboth[half]
            nch = (count + LCH - 1) // LCH

            def zr(r, _):
                for q in range(H // 16):
                    sl = pl.ds(q * 16, 16)
                    sacc[r, sl] = zf
                    gacc[r, sl] = zf
                    macc[r, sl] = negf
                return 0
            lax.fori_loop(0, 168, zr, 0)

            def chunk_body(ch, _):
                cbase = pl.multiple_of(
                    wid * CAP + half * CAP2 + ch * LCH, 8)
                pltpu.sync_copy(lsrc.at[pl.ds(cbase, LCH)], lsb)
                pltpu.sync_copy(ldst.at[pl.ds(cbase, LCH)], ldb)
                pltpu.sync_copy(lew.at[pl.ds(cbase, LCH)], leb)
                rem = count - ch * LCH
                nsub_c = jnp.minimum(LCH // SUBD,
                                     (rem + SUBD - 1) // SUBD)

                def sub(i, _2):
                    pltpu.async_copy(
                        h_hbm.at[lsb.at[pl.ds(i * SUBD, SUBD)]],
                        rows, sem).wait()

                    def grp(g, _3):
                        sl16 = pl.ds(i * SUBD + g * 16, 16)
                        d16 = ldb[sl16]
                        s16 = lsb[sl16]
                        e16 = leb[sl16]
                        q16 = (e16 * plsc.load_gather(isqt, [s16])
                               * plsc.load_gather(
                                   isqt, [jnp.minimum(d16, N2 - 1)]))
                        for lane in range(16):
                            e = g * 16 + lane
                            dl = jnp.minimum(d16[lane] - h_lo, 160)
                            w = e16[lane]
                            qq = q16[lane]
                            for qs in range(H // 16):
                                sl = pl.ds(qs * 16, 16)
                                r = rows[e, sl]
                                sacc[dl, sl] = sacc[dl, sl] + r * w
                                gacc[dl, sl] = gacc[dl, sl] + r * qq
                                macc[dl, sl] = jnp.maximum(macc[dl, sl], r)
                        return 0
                    lax.fori_loop(0, SUBD // 16, grp, 0)
                    return 0
                lax.fori_loop(0, nsub_c, sub, 0)
                return 0
            lax.fori_loop(0, nch, chunk_body, 0)

            def fx(r, _):
                for q in range(H // 16):
                    sl = pl.ds(q * 16, 16)
                    v = macc[r, sl]
                    macc[r, sl] = jnp.where(v < -1.0e38, 0.0, v)
                return 0
            lax.fori_loop(0, HLOC, fx, 0)
            obase = pl.multiple_of(h_lo, 8)
            pltpu.sync_copy(sacc.at[pl.ds(0, HLOC)],
                            sum_out.at[pl.ds(obase, HLOC)])
            pltpu.sync_copy(gacc.at[pl.ds(0, HLOC)],
                            gcn_out.at[pl.ds(obase, HLOC)])
            pltpu.sync_copy(macc.at[pl.ds(0, HLOC)],
                            mx_out.at[pl.ds(obase, HLOC)])

    return kern


def _make_kernel_p():
    BP = 2000

    @functools.partial(
        pl.kernel,
        out_type=[jax.ShapeDtypeStruct((NW, 4, H), jnp.float32)],
        mesh=_sc_mesh(),
        compiler_params=_params_sc,
        scratch_types=[
            pltpu.VMEM((BP,), jnp.int32),
            pltpu.VMEM((BP + SUB,), jnp.int32),
            pltpu.VMEM((BP + SUB,), jnp.int32),
            pltpu.VMEM((8, H), jnp.float32),
            pltpu.VMEM((8, H), jnp.float32),
            pltpu.VMEM((SUB, H), jnp.float32),
            pltpu.VMEM((16,), jnp.float32),
            pltpu.VMEM((8, H), jnp.float32),
            pltpu.SMEM((8,), jnp.int32),
            pltpu.SemaphoreType.DMA,
        ],
    )
    def kern(merge_hbm, batch_hbm, pw_hbm, ro_out,
             bbuf, nid, gid, psum, pmax, rows, pwv, ost, cnt_s, sem):
        cid, sid, wid = _widx()
        glo = wid * 4
        iota = lax.iota(jnp.int32, 16)
        zf = jnp.zeros((16,), jnp.float32)
        negf = jnp.full((16,), NEG, jnp.float32)
        for r in range(8):
            for q in range(H // 16):
                psum[r, pl.ds(q * 16, 16)] = zf
                pmax[r, pl.ds(q * 16, 16)] = negf
            cnt_s[r] = 0
        pltpu.sync_copy(pw_hbm, pwv)

        for b5 in range(N // BP):
            pltpu.sync_copy(batch_hbm.at[pl.ds(b5 * BP, BP)], bbuf)

            def vb(v, cur):
                sl = pl.ds(v * 16, 16)
                b16 = bbuf[sl]
                m = (b16 >= glo) & (b16 < glo + 4)
                mi = m.astype(jnp.int32)
                pos = cur + plsc.cumsum(mi) - mi
                plsc.store_scatter(nid, [pos], b5 * BP + v * 16 + iota, mask=m)
                plsc.store_scatter(gid, [pos], b16 - glo, mask=m)
                return cur + jnp.max(plsc.all_reduce_population_count(m))
            cursor = lax.fori_loop(0, BP // 16, vb, jnp.int32(0))

            nsub = (cursor + SUB - 1) // SUB
            tbase = (cursor // 16) * 16
            for g in range(SUB // 16 + 1):
                idx = tbase + g * 16 + iota
                mm = (idx >= cursor) & (idx < nsub * SUB)
                plsc.store_scatter(nid, [idx], jnp.zeros((16,), jnp.int32),
                                   mask=mm)
                plsc.store_scatter(gid, [idx], jnp.full((16,), 4, jnp.int32),
                                   mask=mm)

            def sub_body(i, _):
                pltpu.async_copy(merge_hbm.at[nid.at[pl.ds(i * SUB, SUB)]],
                                 rows, sem).wait()

                def grp(g, _2):
                    g16 = gid[pl.ds(i * SUB + g * 16, 16)]
                    for lane in range(16):
                        e = g * 16 + lane
                        gl = jnp.minimum(g16[lane], 4)
                        cnt_s[gl] = cnt_s[gl] + 1
                        for qs in range(H // 16):
                            sl = pl.ds(qs * 16, 16)
                            r = rows[e, sl]
                            psum[gl, sl] = psum[gl, sl] + r
                            pmax[gl, sl] = jnp.maximum(pmax[gl, sl], r)
                    return 0
                lax.fori_loop(0, SUB // 16, grp, 0)
                return 0
            lax.fori_loop(0, nsub, sub_body, 0)

        pw = pwv[...]
        for gg in range(4):
            c = jnp.full((16,), cnt_s[gg], jnp.int32).astype(jnp.float32)
            coef = pw[0] * _recip16(jnp.maximum(c, 1.0)) + pw[1]
            for qs in range(H // 16):
                sl = pl.ds(qs * 16, 16)
                mx = pmax[gg, sl]
                mx = jnp.where(mx < -1.0e38, 0.0, mx)
                ost[gg, sl] = coef * psum[gg, sl] + pw[2] * mx
        pltpu.sync_copy(ost.at[pl.ds(0, 4)], ro_out.at[wid])

    return kern


_KS = None


def _get_kernels():
    global _KS
    if _KS is None:
        _KS = (_make_kernel_a(), _make_kernel_d(), _make_kernel_p())
    return _KS


def kernel(x, edge_weight, h_info_node, lin1_W, lin1_b, W_na,
           na_alphas, sc_alphas, la_alphas, pool_alphas, edge_index, batch):
    ka, kd, kp = _get_kernels()
    L = W_na.shape[0]
    na_w = jax.nn.softmax(na_alphas, axis=-1)
    sc_w = jax.nn.softmax(sc_alphas, axis=-1)
    la_w = jax.nn.softmax(la_alphas, axis=-1)
    pool_w = jax.nn.softmax(pool_alphas, axis=-1)
    src = edge_index[0]
    dst = edge_index[1]

    x_pad = jnp.pad(x, ((0, N2 - N), (0, 0)))
    b2d = jnp.broadcast_to(lin1_b[None, :], (8, H))

    lsrc, lew, ldst, cnts, isq, invc = ka(src, dst, edge_weight)
    h = _lin1(x_pad, lin1_W, b2d)
    invc2d = jnp.broadcast_to(invc[:, None], (N2, H))

    w0 = na_w[0:1]
    msc = jnp.concatenate([sc_w[0, 1:2], sc_w[1, 1:2], la_w[0],
                           jnp.zeros((3,), jnp.float32)])[None, :]

    hs = []
    for l in range(L):
        sum_agg, gcn_agg, max_agg = kd(h, lsrc, lew, ldst, cnts, isq)
        if l < L - 1:
            h = _layer(h, sum_agg, max_agg, gcn_agg, invc2d, W_na[l], w0)
            hs.append(h)
        else:
            merge = _layer3(h, sum_agg, max_agg, gcn_agg, invc2d, W_na[l],
                            w0, hs[0], hs[1], msc)

    pw = jnp.concatenate([pool_w[0], jnp.zeros((13,), jnp.float32)])
    ro = kp(merge, batch, pw)[0]
    return ro.reshape(G, H)


# speedup vs baseline: 5.0508x; 1.1039x over previous
import functools
import jax
import jax.numpy as jnp
from jax import lax
from jax.experimental import pallas as pl
from jax.experimental.pallas import tpu as pltpu
from jax.experimental.pallas import tpu_sc as plsc

N = 10000
N2 = 10240
E = 320000
H = 128
G = 128
NW = 32
NLOC = N2 // NW
CAP = 16384
BSCAN = 6400
SUB = 128
CAP2 = CAP // 2
HLOC = 160
ACC_ROWS = 5248
DUMP = 5120
NEG = -3.0e38
BLK = 512

_params_sc = pltpu.CompilerParams(needs_layout_passes=False)


def _widx():
    cid = lax.axis_index("c")
    sid = lax.axis_index("s")
    return cid, sid, cid * 16 + sid


def _sc_mesh():
    return plsc.VectorSubcoreMesh(core_axis_name="c", subcore_axis_name="s")


def _rsqrt16(d):
    y = plsc.bitcast(
        jnp.int32(0x5F3759DF) - (plsc.bitcast(d, jnp.int32) >> 1),
        jnp.float32)
    for _i in range(4):
        y = y * (1.5 - 0.5 * d * y * y)
    return y


def _recip16(x):
    y = _rsqrt16(x)
    return y * y


def _make_kernel_a():
    @functools.partial(
        pl.kernel,
        out_type=[
            jax.ShapeDtypeStruct((NW * CAP,), jnp.int32),
            jax.ShapeDtypeStruct((NW * CAP,), jnp.float32),
            jax.ShapeDtypeStruct((NW * CAP,), jnp.int32),
            jax.ShapeDtypeStruct((NW * 16,), jnp.int32),
            jax.ShapeDtypeStruct((N2,), jnp.float32),
            jax.ShapeDtypeStruct((N2,), jnp.float32),
        ],
        mesh=_sc_mesh(),
        compiler_params=_params_sc,
        scratch_types=[
            pltpu.VMEM((BSCAN,), jnp.int32),
            pltpu.VMEM((BSCAN,), jnp.int32),
            pltpu.VMEM((BSCAN,), jnp.float32),
            pltpu.VMEM((CAP,), jnp.int32),
            pltpu.VMEM((CAP,), jnp.float32),
            pltpu.VMEM((CAP,), jnp.int32),
            pltpu.VMEM((CAP,), jnp.int32),
            pltpu.VMEM((CAP,), jnp.float32),
            pltpu.VMEM((CAP,), jnp.int32),
            pltpu.VMEM((48, H), jnp.float32),
            pltpu.VMEM((NLOC,), jnp.float32),
            pltpu.VMEM((NLOC,), jnp.float32),
            pltpu.VMEM((16,), jnp.int32),
        ],
    )
    def kern(src_hbm, dst_hbm, ew_hbm,
             lsrc_out, lew_out, ldst_out, cnt_out, isq_out, invc_out,
             sbuf, dbuf, ebuf, ls, le, ld, ls2, le2, ld2, dacc, isq_st, invc_st, c16):
        cid, sid, wid = _widx()
        lo = wid * NLOC
        iota = lax.iota(jnp.int32, 16)
        zf = jnp.zeros((16,), jnp.float32)
        zi = jnp.zeros((16,), jnp.int32)

        def zrow(r, _):
            for q in range(H // 16):
                dacc[r, pl.ds(q * 16, 16)] = zf
            return 0
        lax.fori_loop(0, 48, zrow, 0)

        def blk_body(blk, cursor):
            off = pl.multiple_of(blk * BSCAN, 8)
            pltpu.sync_copy(src_hbm.at[pl.ds(off, BSCAN)], sbuf)
            pltpu.sync_copy(dst_hbm.at[pl.ds(off, BSCAN)], dbuf)
            pltpu.sync_copy(ew_hbm.at[pl.ds(off, BSCAN)], ebuf)

            def vreg_body(v, cur):
                sl = pl.ds(v * 16, 16)
                d16 = dbuf[sl]
                s16 = sbuf[sl]
                e16 = ebuf[sl]
                m = (d16 >= lo) & (d16 < lo + NLOC)
                mi = m.astype(jnp.int32)
                pos = cur + plsc.cumsum(mi) - mi
                pos = jnp.minimum(pos, CAP - 1)
                plsc.store_scatter(ls, [pos], s16, mask=m)
                plsc.store_scatter(le, [pos], e16, mask=m)
                plsc.store_scatter(ld, [pos], d16, mask=m)
                return cur + jnp.max(plsc.all_reduce_population_count(m))
            return lax.fori_loop(0, BSCAN // 16, vreg_body, cursor)
        cursor = lax.fori_loop(0, E // BSCAN, blk_body, jnp.int32(0))
        count = jnp.minimum(cursor, CAP)

        mid = lo + HLOC

        def rp_body(v, carry):
            c0, c1 = carry
            sl = pl.ds(v * 16, 16)
            d16 = ld[sl]
            s16 = ls[sl]
            e16 = le[sl]
            valid = (v * 16 + iota) < count
            m0 = valid & (d16 < mid)
            m1 = valid & (d16 >= mid)
            m0i = m0.astype(jnp.int32)
            m1i = m1.astype(jnp.int32)
            pos0 = jnp.minimum(c0 + plsc.cumsum(m0i) - m0i, CAP2 - 1)
            pos1 = jnp.minimum(c1 + plsc.cumsum(m1i) - m1i, CAP2 - 1) + CAP2
            plsc.store_scatter(ls2, [pos0], s16, mask=m0)
            plsc.store_scatter(le2, [pos0], e16, mask=m0)
            plsc.store_scatter(ld2, [pos0], d16, mask=m0)
            plsc.store_scatter(ls2, [pos1], s16, mask=m1)
            plsc.store_scatter(le2, [pos1], e16, mask=m1)
            plsc.store_scatter(ld2, [pos1], d16, mask=m1)
            c0 = c0 + jnp.max(plsc.all_reduce_population_count(m0))
            c1 = c1 + jnp.max(plsc.all_reduce_population_count(m1))
            return c0, c1
        c0, c1 = lax.fori_loop(0, (count + 15) // 16, rp_body,
                               (jnp.int32(0), jnp.int32(0)))
        c0 = jnp.minimum(c0, CAP2)
        c1 = jnp.minimum(c1, CAP2)

        dumpdst = jnp.full((16,), cid * 5120 + DUMP, jnp.int32)
        for half, ch in ((0, c0), (1, c1)):
            nsub_h = (ch + SUB - 1) // SUB
            base_h = (ch // 16) * 16
            for g in range(SUB // 16 + 1):
                idx = base_h + g * 16 + iota
                mm = (idx >= ch) & (idx < nsub_h * SUB)
                idxc = jnp.minimum(idx, CAP2 - 1) + half * CAP2
                plsc.store_scatter(ls2, [idxc], zi, mask=mm)
                plsc.store_scatter(le2, [idxc], zf, mask=mm)
                plsc.store_scatter(ld2, [idxc], dumpdst, mask=mm)

        one16 = (iota == 1).astype(jnp.float32)
        def grp_body(g, _):
            sl = pl.ds(g * 16, 16)
            d16 = ld[sl]
            e16 = le[sl]
            for lane in range(16):
                valid = (g * 16 + lane) < count
                dl = jnp.clip(d16[lane] - lo, 0, 320)
                row = dl >> 3
                col = (dl & 7) * 16
                sl2 = pl.ds(col, 16)
                mini = jnp.where(iota == 0, e16[lane], one16)
                mini = jnp.where(valid, mini, 0.0)
                dacc[row, sl2] = dacc[row, sl2] + mini
            return 0
        lax.fori_loop(0, (count + 15) // 16, grp_body, 0)

        def nb_body(nb, _):
            r16 = nb * 16 + iota
            row16 = r16 >> 3
            col16 = (r16 & 7) * 16
            deg = plsc.load_gather(dacc, [row16, col16])
            cnt = plsc.load_gather(dacc, [row16, col16 + 1])
            isq = jnp.where(deg > 0, _rsqrt16(jnp.maximum(deg, 1e-12)), 0.0)
            isq_st[pl.ds(nb * 16, 16)] = isq
            invc_st[pl.ds(nb * 16, 16)] = _recip16(jnp.maximum(cnt, 1.0))
            return 0
        lax.fori_loop(0, NLOC // 16, nb_body, 0)

        pltpu.sync_copy(ls2, lsrc_out.at[pl.ds(wid * CAP, CAP)])
        pltpu.sync_copy(le2, lew_out.at[pl.ds(wid * CAP, CAP)])
        pltpu.sync_copy(ld2, ldst_out.at[pl.ds(wid * CAP, CAP)])
        iota16 = lax.iota(jnp.int32, 16)
        c16[...] = jnp.where(iota16 == 0, c0, jnp.where(iota16 == 1, c1, 0))
        pltpu.sync_copy(c16, cnt_out.at[pl.ds(wid * 16, 16)])
        pltpu.sync_copy(isq_st, isq_out.at[pl.ds(wid * NLOC, NLOC)])
        pltpu.sync_copy(invc_st, invc_out.at[pl.ds(wid * NLOC, NLOC)])

    return kern


def _elu(v):
    return jnp.where(v > 0, v, jnp.exp(jnp.minimum(v, 0.0)) - 1.0)


def _lin1_body(x_ref, W_ref, b_ref, out_ref):
    out_ref[...] = (jnp.dot(x_ref[...], W_ref[...],
                            preferred_element_type=jnp.float32)
                    + b_ref[...][0:1, :])


@jax.jit
def _lin1(x_pad, W, b2d):
    row = pl.BlockSpec((BLK, H), lambda i: (i, 0))
    return pl.pallas_call(
        _lin1_body,
        grid=(N2 // BLK,),
        in_specs=[row, pl.BlockSpec((H, H), lambda i: (0, 0)),
                  pl.BlockSpec((8, H), lambda i: (0, 0))],
        out_specs=row,
        out_shape=jax.ShapeDtypeStruct((N2, H), jnp.float32),
    )(x_pad, W, b2d)


def _layer_body(h_ref, sum_ref, max_ref, gcn_ref, ic_ref, W_ref, w_ref,
                out_ref):
    h = h_ref[...]
    W = W_ref[...]
    s = sum_ref[...]
    o0 = _elu(jnp.dot(gcn_ref[...], W[0], preferred_element_type=jnp.float32))
    o1 = _elu(jnp.dot(h + s * ic_ref[...], W[1],
                      preferred_element_type=jnp.float32))
    o2 = _elu(jnp.dot(h + s, W[2], preferred_element_type=jnp.float32))
    o3 = _elu(jnp.dot(max_ref[...], W[3], preferred_element_type=jnp.float32))
    o4 = jnp.dot(h, W[4], preferred_element_type=jnp.float32)
    out_ref[...] = (w_ref[0, 0] * o0 + w_ref[0, 1] * o1 + w_ref[0, 2] * o2
                    + w_ref[0, 3] * o3 + w_ref[0, 4] * o4)


@jax.jit
def _layer(h, sum_agg, max_agg, gcn_agg, invc2d, W_l, w):
    row = pl.BlockSpec((BLK, H), lambda i: (i, 0))
    return pl.pallas_call(
        _layer_body,
        grid=(N2 // BLK,),
        in_specs=[row, row, row, row, row,
                  pl.BlockSpec((5, H, H), lambda i: (0, 0, 0)),
                  pl.BlockSpec((1, 5), lambda i: (0, 0),
                               memory_space=pltpu.SMEM)],
        out_specs=row,
        out_shape=jax.ShapeDtypeStruct((N2, H), jnp.float32),
    )(h, sum_agg, max_agg, gcn_agg, invc2d, W_l, w)


def _layer3_body(h_ref, sum_ref, max_ref, gcn_ref, ic_ref, W_ref, w_ref,
                 h1_ref, h2_ref, m_ref, out_ref):
    h = h_ref[...]
    W = W_ref[...]
    s = sum_ref[...]
    o0 = _elu(jnp.dot(gcn_ref[...], W[0], preferred_element_type=jnp.float32))
    o1 = _elu(jnp.dot(h + s * ic_ref[...], W[1],
                      preferred_element_type=jnp.float32))
    o2 = _elu(jnp.dot(h + s, W[2], preferred_element_type=jnp.float32))
    o3 = _elu(jnp.dot(max_ref[...], W[3], preferred_element_type=jnp.float32))
    o4 = jnp.dot(h, W[4], preferred_element_type=jnp.float32)
    h3 = (w_ref[0, 0] * o0 + w_ref[0, 1] * o1 + w_ref[0, 2] * o2
          + w_ref[0, 3] * o3 + w_ref[0, 4] * o4)
    j1 = m_ref[0, 0] * h1_ref[...]
    j2 = m_ref[0, 1] * h2_ref[...]
    tot = j1 + j2 + h3
    mx = jnp.maximum(jnp.maximum(j1, j2), h3)
    out_ref[...] = ((m_ref[0, 2] + m_ref[0, 3] / 3.0) * tot
                    + m_ref[0, 4] * mx)


@jax.jit
def _layer3(h, sum_agg, max_agg, gcn_agg, invc2d, W_l, w, h1, h2, msc):
    row = pl.BlockSpec((BLK, H), lambda i: (i, 0))
    return pl.pallas_call(
        _layer3_body,
        grid=(N2 // BLK,),
        in_specs=[row, row, row, row, row,
                  pl.BlockSpec((5, H, H), lambda i: (0, 0, 0)),
                  pl.BlockSpec((1, 5), lambda i: (0, 0),
                               memory_space=pltpu.SMEM),
                  row, row,
                  pl.BlockSpec((1, 8), lambda i: (0, 0),
                               memory_space=pltpu.SMEM)],
        out_specs=row,
        out_shape=jax.ShapeDtypeStruct((N2, H), jnp.float32),
    )(h, sum_agg, max_agg, gcn_agg, invc2d, W_l, w, h1, h2, msc)


SUBD = 128
LCH = 1024


def _make_kernel_d():
    @functools.partial(
        pl.kernel,
        out_type=[
            jax.ShapeDtypeStruct((N2, H), jnp.float32),
            jax.ShapeDtypeStruct((N2, H), jnp.float32),
            jax.ShapeDtypeStruct((N2, H), jnp.float32),
        ],
        mesh=_sc_mesh(),
        compiler_params=_params_sc,
        scratch_types=[
            pltpu.VMEM((168, H), jnp.float32),
            pltpu.VMEM((168, H), jnp.float32),
            pltpu.VMEM((168, H), jnp.float32),
            pltpu.VMEM((N2,), jnp.float32),
            pltpu.VMEM((LCH,), jnp.int32),
            pltpu.VMEM((LCH,), jnp.int32),
            pltpu.VMEM((LCH,), jnp.float32),
            pltpu.VMEM((2, SUBD, H), jnp.float32),
            pltpu.VMEM((16,), jnp.int32),
            pltpu.SemaphoreType.DMA,
        ],
    )
    def kern(h_hbm, lsrc, lew, ldst, cnts, isq_hbm,
             sum_out, gcn_out, mx_out,
             sacc, gacc, macc, isqt, lsb, ldb, leb, rows, c16, sem):
        cid, sid, wid = _widx()
        lo = wid * NLOC
        zf = jnp.zeros((16,), jnp.float32)
        negf = jnp.full((16,), NEG, jnp.float32)
        pltpu.sync_copy(isq_hbm, isqt)
        pltpu.sync_copy(cnts.at[pl.ds(wid * 16, 16)], c16)
        cboth = c16[...]

        for half in range(2):
            h_lo = lo + half * HLOC
            count = cboth[half]
            nch = (count + LCH - 1) // LCH

            def zr(r, _):
                for q in range(H // 16):
                    sl = pl.ds(q * 16, 16)
                    sacc[r, sl] = zf
                    gacc[r, sl] = zf
                    macc[r, sl] = negf
                return 0
            lax.fori_loop(0, 168, zr, 0)

            def chunk_body(ch, _):
                cbase = pl.multiple_of(
                    wid * CAP + half * CAP2 + ch * LCH, 8)
                pltpu.sync_copy(lsrc.at[pl.ds(cbase, LCH)], lsb)
                pltpu.sync_copy(ldst.at[pl.ds(cbase, LCH)], ldb)
                pltpu.sync_copy(lew.at[pl.ds(cbase, LCH)], leb)
                rem = count - ch * LCH
                nsub_c = jnp.minimum(LCH // SUBD,
                                     (rem + SUBD - 1) // SUBD)

                pltpu.async_copy(
                    h_hbm.at[lsb.at[pl.ds(0, SUBD)]], rows.at[0], sem)

                def sub(i, _2):
                    b = i % 2

                    @pl.when(i + 1 < nsub_c)
                    def _prefetch():
                        nxt = pl.ds((i + 1) * SUBD, SUBD)
                        pltpu.async_copy(h_hbm.at[lsb.at[nxt]],
                                         rows.at[(i + 1) % 2], sem)

                    pltpu.make_async_copy(
                        h_hbm.at[lsb.at[pl.ds(i * SUBD, SUBD)]],
                        rows.at[b], sem).wait()

                    def grp(g, _3):
                        sl16 = pl.ds(i * SUBD + g * 16, 16)
                        d16 = ldb[sl16]
                        s16 = lsb[sl16]
                        e16 = leb[sl16]
                        q16 = (e16 * plsc.load_gather(isqt, [s16])
                               * plsc.load_gather(
                                   isqt, [jnp.minimum(d16, N2 - 1)]))
                        for lane in range(16):
                            e = g * 16 + lane
                            dl = jnp.minimum(d16[lane] - h_lo, 160)
                            w = e16[lane]
                            qq = q16[lane]
                            for qs in range(H // 16):
                                sl = pl.ds(qs * 16, 16)
                                r = rows[b, e, sl]
                                sacc[dl, sl] = sacc[dl, sl] + r * w
                                gacc[dl, sl] = gacc[dl, sl] + r * qq
                                macc[dl, sl] = jnp.maximum(macc[dl, sl], r)
                        return 0
                    lax.fori_loop(0, SUBD // 16, grp, 0)
                    return 0
                lax.fori_loop(0, nsub_c, sub, 0)
                return 0
            lax.fori_loop(0, nch, chunk_body, 0)

            def fx(r, _):
                for q in range(H // 16):
                    sl = pl.ds(q * 16, 16)
                    v = macc[r, sl]
                    macc[r, sl] = jnp.where(v < -1.0e38, 0.0, v)
                return 0
            lax.fori_loop(0, HLOC, fx, 0)
            obase = pl.multiple_of(h_lo, 8)
            pltpu.sync_copy(sacc.at[pl.ds(0, HLOC)],
                            sum_out.at[pl.ds(obase, HLOC)])
            pltpu.sync_copy(gacc.at[pl.ds(0, HLOC)],
                            gcn_out.at[pl.ds(obase, HLOC)])
            pltpu.sync_copy(macc.at[pl.ds(0, HLOC)],
                            mx_out.at[pl.ds(obase, HLOC)])

    return kern


def _make_kernel_p():
    BP = 2000

    @functools.partial(
        pl.kernel,
        out_type=[jax.ShapeDtypeStruct((NW, 4, H), jnp.float32)],
        mesh=_sc_mesh(),
        compiler_params=_params_sc,
        scratch_types=[
            pltpu.VMEM((BP,), jnp.int32),
            pltpu.VMEM((BP + SUB,), jnp.int32),
            pltpu.VMEM((BP + SUB,), jnp.int32),
            pltpu.VMEM((8, H), jnp.float32),
            pltpu.VMEM((8, H), jnp.float32),
            pltpu.VMEM((SUB, H), jnp.float32),
            pltpu.VMEM((16,), jnp.float32),
            pltpu.VMEM((8, H), jnp.float32),
            pltpu.SMEM((8,), jnp.int32),
            pltpu.SemaphoreType.DMA,
        ],
    )
    def kern(merge_hbm, batch_hbm, pw_hbm, ro_out,
             bbuf, nid, gid, psum, pmax, rows, pwv, ost, cnt_s, sem):
        cid, sid, wid = _widx()
        glo = wid * 4
        iota = lax.iota(jnp.int32, 16)
        zf = jnp.zeros((16,), jnp.float32)
        negf = jnp.full((16,), NEG, jnp.float32)
        for r in range(8):
            for q in range(H // 16):
                psum[r, pl.ds(q * 16, 16)] = zf
                pmax[r, pl.ds(q * 16, 16)] = negf
            cnt_s[r] = 0
        pltpu.sync_copy(pw_hbm, pwv)

        for b5 in range(N // BP):
            pltpu.sync_copy(batch_hbm.at[pl.ds(b5 * BP, BP)], bbuf)

            def vb(v, cur):
                sl = pl.ds(v * 16, 16)
                b16 = bbuf[sl]
                m = (b16 >= glo) & (b16 < glo + 4)
                mi = m.astype(jnp.int32)
                pos = cur + plsc.cumsum(mi) - mi
                plsc.store_scatter(nid, [pos], b5 * BP + v * 16 + iota, mask=m)
                plsc.store_scatter(gid, [pos], b16 - glo, mask=m)
                return cur + jnp.max(plsc.all_reduce_population_count(m))
            cursor = lax.fori_loop(0, BP // 16, vb, jnp.int32(0))

            nsub = (cursor + SUB - 1) // SUB
            tbase = (cursor // 16) * 16
            for g in range(SUB // 16 + 1):
                idx = tbase + g * 16 + iota
                mm = (idx >= cursor) & (idx < nsub * SUB)
                plsc.store_scatter(nid, [idx], jnp.zeros((16,), jnp.int32),
                                   mask=mm)
                plsc.store_scatter(gid, [idx], jnp.full((16,), 4, jnp.int32),
                                   mask=mm)

            def sub_body(i, _):
                pltpu.async_copy(merge_hbm.at[nid.at[pl.ds(i * SUB, SUB)]],
                                 rows, sem).wait()

                def grp(g, _2):
                    g16 = gid[pl.ds(i * SUB + g * 16, 16)]
                    for lane in range(16):
                        e = g * 16 + lane
                        gl = jnp.minimum(g16[lane], 4)
                        cnt_s[gl] = cnt_s[gl] + 1
                        for qs in range(H // 16):
                            sl = pl.ds(qs * 16, 16)
                            r = rows[e, sl]
                            psum[gl, sl] = psum[gl, sl] + r
                            pmax[gl, sl] = jnp.maximum(pmax[gl, sl], r)
                    return 0
                lax.fori_loop(0, SUB // 16, grp, 0)
                return 0
            lax.fori_loop(0, nsub, sub_body, 0)

        pw = pwv[...]
        for gg in range(4):
            c = jnp.full((16,), cnt_s[gg], jnp.int32).astype(jnp.float32)
            coef = pw[0] * _recip16(jnp.maximum(c, 1.0)) + pw[1]
            for qs in range(H // 16):
                sl = pl.ds(qs * 16, 16)
                mx = pmax[gg, sl]
                mx = jnp.where(mx < -1.0e38, 0.0, mx)
                ost[gg, sl] = coef * psum[gg, sl] + pw[2] * mx
        pltpu.sync_copy(ost.at[pl.ds(0, 4)], ro_out.at[wid])

    return kern


_KS = None


def _get_kernels():
    global _KS
    if _KS is None:
        _KS = (_make_kernel_a(), _make_kernel_d(), _make_kernel_p())
    return _KS


def kernel(x, edge_weight, h_info_node, lin1_W, lin1_b, W_na,
           na_alphas, sc_alphas, la_alphas, pool_alphas, edge_index, batch):
    ka, kd, kp = _get_kernels()
    L = W_na.shape[0]
    na_w = jax.nn.softmax(na_alphas, axis=-1)
    sc_w = jax.nn.softmax(sc_alphas, axis=-1)
    la_w = jax.nn.softmax(la_alphas, axis=-1)
    pool_w = jax.nn.softmax(pool_alphas, axis=-1)
    src = edge_index[0]
    dst = edge_index[1]

    x_pad = jnp.pad(x, ((0, N2 - N), (0, 0)))
    b2d = jnp.broadcast_to(lin1_b[None, :], (8, H))

    lsrc, lew, ldst, cnts, isq, invc = ka(src, dst, edge_weight)
    h = _lin1(x_pad, lin1_W, b2d)
    invc2d = jnp.broadcast_to(invc[:, None], (N2, H))

    w0 = na_w[0:1]
    msc = jnp.concatenate([sc_w[0, 1:2], sc_w[1, 1:2], la_w[0],
                           jnp.zeros((3,), jnp.float32)])[None, :]

    hs = []
    for l in range(L):
        sum_agg, gcn_agg, max_agg = kd(h, lsrc, lew, ldst, cnts, isq)
        if l < L - 1:
            h = _layer(h, sum_agg, max_agg, gcn_agg, invc2d, W_na[l], w0)
            hs.append(h)
        else:
            merge = _layer3(h, sum_agg, max_agg, gcn_agg, invc2d, W_na[l],
                            w0, hs[0], hs[1], msc)

    pw = jnp.concatenate([pool_w[0], jnp.zeros((13,), jnp.float32)])
    ro = kp(merge, batch, pw)[0]
    return ro.reshape(G, H)


# speedup vs baseline: 5.2495x; 1.0393x over previous
import functools
import jax
import jax.numpy as jnp
from jax import lax
from jax.experimental import pallas as pl
from jax.experimental.pallas import tpu as pltpu
from jax.experimental.pallas import tpu_sc as plsc

N = 10000
N2 = 10240
E = 320000
H = 128
G = 128
NW = 32
NLOC = N2 // NW
CAP = 16384
BSCAN = 6400
SUB = 128
CAP2 = CAP // 2
HLOC = 160
ACC_ROWS = 5248
DUMP = 5120
NEG = -3.0e38
BLK = 512

_params_sc = pltpu.CompilerParams(needs_layout_passes=False)


def _widx():
    cid = lax.axis_index("c")
    sid = lax.axis_index("s")
    return cid, sid, cid * 16 + sid


def _sc_mesh():
    return plsc.VectorSubcoreMesh(core_axis_name="c", subcore_axis_name="s")


def _rsqrt16(d):
    y = plsc.bitcast(
        jnp.int32(0x5F3759DF) - (plsc.bitcast(d, jnp.int32) >> 1),
        jnp.float32)
    for _i in range(4):
        y = y * (1.5 - 0.5 * d * y * y)
    return y


def _recip16(x):
    y = _rsqrt16(x)
    return y * y


def _make_kernel_a():
    @functools.partial(
        pl.kernel,
        out_type=[
            jax.ShapeDtypeStruct((NW * CAP,), jnp.int32),
            jax.ShapeDtypeStruct((NW * CAP,), jnp.float32),
            jax.ShapeDtypeStruct((NW * CAP,), jnp.int32),
            jax.ShapeDtypeStruct((NW * 16,), jnp.int32),
            jax.ShapeDtypeStruct((N2,), jnp.float32),
            jax.ShapeDtypeStruct((N2,), jnp.float32),
        ],
        mesh=_sc_mesh(),
        compiler_params=_params_sc,
        scratch_types=[
            pltpu.VMEM((BSCAN,), jnp.int32),
            pltpu.VMEM((BSCAN,), jnp.int32),
            pltpu.VMEM((BSCAN,), jnp.float32),
            pltpu.VMEM((CAP,), jnp.int32),
            pltpu.VMEM((CAP,), jnp.float32),
            pltpu.VMEM((CAP,), jnp.int32),
            pltpu.VMEM((CAP,), jnp.int32),
            pltpu.VMEM((CAP,), jnp.float32),
            pltpu.VMEM((CAP,), jnp.int32),
            pltpu.VMEM((48, H), jnp.float32),
            pltpu.VMEM((NLOC,), jnp.float32),
            pltpu.VMEM((NLOC,), jnp.float32),
            pltpu.VMEM((16,), jnp.int32),
            pltpu.SemaphoreType.DMA,
            pltpu.SemaphoreType.DMA,
            pltpu.SemaphoreType.DMA,
        ],
    )
    def kern(src_hbm, dst_hbm, ew_hbm,
             lsrc_out, lew_out, ldst_out, cnt_out, isq_out, invc_out,
             sbuf, dbuf, ebuf, ls, le, ld, ls2, le2, ld2, dacc, isq_st, invc_st, c16,
             sema, semb, semc):
        cid, sid, wid = _widx()
        lo = wid * NLOC
        iota = lax.iota(jnp.int32, 16)
        zf = jnp.zeros((16,), jnp.float32)
        zi = jnp.zeros((16,), jnp.int32)

        def zrow(r, _):
            for q in range(H // 16):
                dacc[r, pl.ds(q * 16, 16)] = zf
            return 0
        lax.fori_loop(0, 48, zrow, 0)

        def blk_body(blk, cursor):
            off = pl.multiple_of(blk * BSCAN, 8)
            h1 = pltpu.async_copy(src_hbm.at[pl.ds(off, BSCAN)], sbuf, sema)
            h2 = pltpu.async_copy(dst_hbm.at[pl.ds(off, BSCAN)], dbuf, semb)
            h3 = pltpu.async_copy(ew_hbm.at[pl.ds(off, BSCAN)], ebuf, semc)
            h1.wait()
            h2.wait()
            h3.wait()

            def vreg_body(v, cur):
                sl = pl.ds(v * 16, 16)
                d16 = dbuf[sl]
                s16 = sbuf[sl]
                e16 = ebuf[sl]
                m = (d16 >= lo) & (d16 < lo + NLOC)
                mi = m.astype(jnp.int32)
                pos = cur + plsc.cumsum(mi) - mi
                pos = jnp.minimum(pos, CAP - 1)
                plsc.store_scatter(ls, [pos], s16, mask=m)
                plsc.store_scatter(le, [pos], e16, mask=m)
                plsc.store_scatter(ld, [pos], d16, mask=m)
                return cur + jnp.max(plsc.all_reduce_population_count(m))
            return lax.fori_loop(0, BSCAN // 16, vreg_body, cursor)
        cursor = lax.fori_loop(0, E // BSCAN, blk_body, jnp.int32(0))
        count = jnp.minimum(cursor, CAP)

        mid = lo + HLOC

        def rp_body(v, carry):
            c0, c1 = carry
            sl = pl.ds(v * 16, 16)
            d16 = ld[sl]
            s16 = ls[sl]
            e16 = le[sl]
            valid = (v * 16 + iota) < count
            m0 = valid & (d16 < mid)
            m1 = valid & (d16 >= mid)
            m0i = m0.astype(jnp.int32)
            m1i = m1.astype(jnp.int32)
            pos0 = jnp.minimum(c0 + plsc.cumsum(m0i) - m0i, CAP2 - 1)
            pos1 = jnp.minimum(c1 + plsc.cumsum(m1i) - m1i, CAP2 - 1) + CAP2
            plsc.store_scatter(ls2, [pos0], s16, mask=m0)
            plsc.store_scatter(le2, [pos0], e16, mask=m0)
            plsc.store_scatter(ld2, [pos0], d16, mask=m0)
            plsc.store_scatter(ls2, [pos1], s16, mask=m1)
            plsc.store_scatter(le2, [pos1], e16, mask=m1)
            plsc.store_scatter(ld2, [pos1], d16, mask=m1)
            c0 = c0 + jnp.max(plsc.all_reduce_population_count(m0))
            c1 = c1 + jnp.max(plsc.all_reduce_population_count(m1))
            return c0, c1
        c0, c1 = lax.fori_loop(0, (count + 15) // 16, rp_body,
                               (jnp.int32(0), jnp.int32(0)))
        c0 = jnp.minimum(c0, CAP2)
        c1 = jnp.minimum(c1, CAP2)

        dumpdst = jnp.full((16,), cid * 5120 + DUMP, jnp.int32)
        for half, ch in ((0, c0), (1, c1)):
            nsub_h = (ch + SUB - 1) // SUB
            base_h = (ch // 16) * 16
            for g in range(SUB // 16 + 1):
                idx = base_h + g * 16 + iota
                mm = (idx >= ch) & (idx < nsub_h * SUB)
                idxc = jnp.minimum(idx, CAP2 - 1) + half * CAP2
                plsc.store_scatter(ls2, [idxc], zi, mask=mm)
                plsc.store_scatter(le2, [idxc], zf, mask=mm)
                plsc.store_scatter(ld2, [idxc], dumpdst, mask=mm)

        one16 = (iota == 1).astype(jnp.float32)
        def grp_body(g, _):
            sl = pl.ds(g * 16, 16)
            d16 = ld[sl]
            e16 = le[sl]
            for lane in range(16):
                valid = (g * 16 + lane) < count
                dl = jnp.clip(d16[lane] - lo, 0, 320)
                row = dl >> 3
                col = (dl & 7) * 16
                sl2 = pl.ds(col, 16)
                mini = jnp.where(iota == 0, e16[lane], one16)
                mini = jnp.where(valid, mini, 0.0)
                dacc[row, sl2] = dacc[row, sl2] + mini
            return 0
        lax.fori_loop(0, (count + 15) // 16, grp_body, 0)

        def nb_body(nb, _):
            r16 = nb * 16 + iota
            row16 = r16 >> 3
            col16 = (r16 & 7) * 16
            deg = plsc.load_gather(dacc, [row16, col16])
            cnt = plsc.load_gather(dacc, [row16, col16 + 1])
            isq = jnp.where(deg > 0, _rsqrt16(jnp.maximum(deg, 1e-12)), 0.0)
            isq_st[pl.ds(nb * 16, 16)] = isq
            invc_st[pl.ds(nb * 16, 16)] = _recip16(jnp.maximum(cnt, 1.0))
            return 0
        lax.fori_loop(0, NLOC // 16, nb_body, 0)

        pltpu.sync_copy(ls2, lsrc_out.at[pl.ds(wid * CAP, CAP)])
        pltpu.sync_copy(le2, lew_out.at[pl.ds(wid * CAP, CAP)])
        pltpu.sync_copy(ld2, ldst_out.at[pl.ds(wid * CAP, CAP)])
        iota16 = lax.iota(jnp.int32, 16)
        c16[...] = jnp.where(iota16 == 0, c0, jnp.where(iota16 == 1, c1, 0))
        pltpu.sync_copy(c16, cnt_out.at[pl.ds(wid * 16, 16)])
        pltpu.sync_copy(isq_st, isq_out.at[pl.ds(wid * NLOC, NLOC)])
        pltpu.sync_copy(invc_st, invc_out.at[pl.ds(wid * NLOC, NLOC)])

    return kern


def _elu(v):
    return jnp.where(v > 0, v, jnp.exp(jnp.minimum(v, 0.0)) - 1.0)


def _lin1_body(x_ref, W_ref, b_ref, out_ref):
    out_ref[...] = (jnp.dot(x_ref[...], W_ref[...],
                            preferred_element_type=jnp.float32)
                    + b_ref[...][0:1, :])


@jax.jit
def _lin1(x_pad, W, b2d):
    row = pl.BlockSpec((BLK, H), lambda i: (i, 0))
    return pl.pallas_call(
        _lin1_body,
        grid=(N2 // BLK,),
        in_specs=[row, pl.BlockSpec((H, H), lambda i: (0, 0)),
                  pl.BlockSpec((8, H), lambda i: (0, 0))],
        out_specs=row,
        out_shape=jax.ShapeDtypeStruct((N2, H), jnp.float32),
    )(x_pad, W, b2d)


def _layer_body(h_ref, sum_ref, max_ref, gcn_ref, ic_ref, W_ref, w_ref,
                out_ref):
    h = h_ref[...]
    W = W_ref[...]
    s = sum_ref[...]
    o0 = _elu(jnp.dot(gcn_ref[...], W[0], preferred_element_type=jnp.float32))
    o1 = _elu(jnp.dot(h + s * ic_ref[...], W[1],
                      preferred_element_type=jnp.float32))
    o2 = _elu(jnp.dot(h + s, W[2], preferred_element_type=jnp.float32))
    o3 = _elu(jnp.dot(max_ref[...], W[3], preferred_element_type=jnp.float32))
    o4 = jnp.dot(h, W[4], preferred_element_type=jnp.float32)
    out_ref[...] = (w_ref[0, 0] * o0 + w_ref[0, 1] * o1 + w_ref[0, 2] * o2
                    + w_ref[0, 3] * o3 + w_ref[0, 4] * o4)


@jax.jit
def _layer(h, sum_agg, max_agg, gcn_agg, invc2d, W_l, w):
    row = pl.BlockSpec((BLK, H), lambda i: (i, 0))
    return pl.pallas_call(
        _layer_body,
        grid=(N2 // BLK,),
        in_specs=[row, row, row, row, row,
                  pl.BlockSpec((5, H, H), lambda i: (0, 0, 0)),
                  pl.BlockSpec((1, 5), lambda i: (0, 0),
                               memory_space=pltpu.SMEM)],
        out_specs=row,
        out_shape=jax.ShapeDtypeStruct((N2, H), jnp.float32),
    )(h, sum_agg, max_agg, gcn_agg, invc2d, W_l, w)


def _layer3_body(h_ref, sum_ref, max_ref, gcn_ref, ic_ref, W_ref, w_ref,
                 h1_ref, h2_ref, m_ref, out_ref):
    h = h_ref[...]
    W = W_ref[...]
    s = sum_ref[...]
    o0 = _elu(jnp.dot(gcn_ref[...], W[0], preferred_element_type=jnp.float32))
    o1 = _elu(jnp.dot(h + s * ic_ref[...], W[1],
                      preferred_element_type=jnp.float32))
    o2 = _elu(jnp.dot(h + s, W[2], preferred_element_type=jnp.float32))
    o3 = _elu(jnp.dot(max_ref[...], W[3], preferred_element_type=jnp.float32))
    o4 = jnp.dot(h, W[4], preferred_element_type=jnp.float32)
    h3 = (w_ref[0, 0] * o0 + w_ref[0, 1] * o1 + w_ref[0, 2] * o2
          + w_ref[0, 3] * o3 + w_ref[0, 4] * o4)
    j1 = m_ref[0, 0] * h1_ref[...]
    j2 = m_ref[0, 1] * h2_ref[...]
    tot = j1 + j2 + h3
    mx = jnp.maximum(jnp.maximum(j1, j2), h3)
    out_ref[...] = ((m_ref[0, 2] + m_ref[0, 3] / 3.0) * tot
                    + m_ref[0, 4] * mx)


@jax.jit
def _layer3(h, sum_agg, max_agg, gcn_agg, invc2d, W_l, w, h1, h2, msc):
    row = pl.BlockSpec((BLK, H), lambda i: (i, 0))
    return pl.pallas_call(
        _layer3_body,
        grid=(N2 // BLK,),
        in_specs=[row, row, row, row, row,
                  pl.BlockSpec((5, H, H), lambda i: (0, 0, 0)),
                  pl.BlockSpec((1, 5), lambda i: (0, 0),
                               memory_space=pltpu.SMEM),
                  row, row,
                  pl.BlockSpec((1, 8), lambda i: (0, 0),
                               memory_space=pltpu.SMEM)],
        out_specs=row,
        out_shape=jax.ShapeDtypeStruct((N2, H), jnp.float32),
    )(h, sum_agg, max_agg, gcn_agg, invc2d, W_l, w, h1, h2, msc)


SUBD = 128
LCH = 1024


def _make_kernel_d():
    @functools.partial(
        pl.kernel,
        out_type=[
            jax.ShapeDtypeStruct((N2, H), jnp.float32),
            jax.ShapeDtypeStruct((N2, H), jnp.float32),
            jax.ShapeDtypeStruct((N2, H), jnp.float32),
        ],
        mesh=_sc_mesh(),
        compiler_params=_params_sc,
        scratch_types=[
            pltpu.VMEM((168, H), jnp.float32),
            pltpu.VMEM((168, H), jnp.float32),
            pltpu.VMEM((168, H), jnp.float32),
            pltpu.VMEM((N2,), jnp.float32),
            pltpu.VMEM((LCH,), jnp.int32),
            pltpu.VMEM((LCH,), jnp.int32),
            pltpu.VMEM((LCH,), jnp.float32),
            pltpu.VMEM((2, SUBD, H), jnp.float32),
            pltpu.VMEM((16,), jnp.int32),
            pltpu.SemaphoreType.DMA,
            pltpu.SemaphoreType.DMA,
            pltpu.SemaphoreType.DMA,
            pltpu.SemaphoreType.DMA,
        ],
    )
    def kern(h_hbm, lsrc, lew, ldst, cnts, isq_hbm,
             sum_out, gcn_out, mx_out,
             sacc, gacc, macc, isqt, lsb, ldb, leb, rows, c16, sem,
             semx, semy, semz):
        cid, sid, wid = _widx()
        lo = wid * NLOC
        zf = jnp.zeros((16,), jnp.float32)
        negf = jnp.full((16,), NEG, jnp.float32)
        pltpu.sync_copy(isq_hbm, isqt)
        pltpu.sync_copy(cnts.at[pl.ds(wid * 16, 16)], c16)
        cboth = c16[...]

        for half in range(2):
            h_lo = lo + half * HLOC
            count = cboth[half]
            nch = (count + LCH - 1) // LCH

            def zr(r, _):
                for q in range(H // 16):
                    sl = pl.ds(q * 16, 16)
                    sacc[r, sl] = zf
                    gacc[r, sl] = zf
                    macc[r, sl] = negf
                return 0
            lax.fori_loop(0, 168, zr, 0)

            def chunk_body(ch, _):
                cbase = pl.multiple_of(
                    wid * CAP + half * CAP2 + ch * LCH, 8)
                h1 = pltpu.async_copy(lsrc.at[pl.ds(cbase, LCH)], lsb, semx)
                h2 = pltpu.async_copy(ldst.at[pl.ds(cbase, LCH)], ldb, semy)
                h3 = pltpu.async_copy(lew.at[pl.ds(cbase, LCH)], leb, semz)
                h1.wait()
                h2.wait()
                h3.wait()
                rem = count - ch * LCH
                nsub_c = jnp.minimum(LCH // SUBD,
                                     (rem + SUBD - 1) // SUBD)

                pltpu.async_copy(
                    h_hbm.at[lsb.at[pl.ds(0, SUBD)]], rows.at[0], sem)

                def sub(i, _2):
                    b = i % 2

                    @pl.when(i + 1 < nsub_c)
                    def _prefetch():
                        nxt = pl.ds((i + 1) * SUBD, SUBD)
                        pltpu.async_copy(h_hbm.at[lsb.at[nxt]],
                                         rows.at[(i + 1) % 2], sem)

                    pltpu.make_async_copy(
                        h_hbm.at[lsb.at[pl.ds(i * SUBD, SUBD)]],
                        rows.at[b], sem).wait()

                    def grp(g, _3):
                        sl16 = pl.ds(i * SUBD + g * 16, 16)
                        d16 = ldb[sl16]
                        s16 = lsb[sl16]
                        e16 = leb[sl16]
                        q16 = (e16 * plsc.load_gather(isqt, [s16])
                               * plsc.load_gather(
                                   isqt, [jnp.minimum(d16, N2 - 1)]))
                        for lane in range(16):
                            e = g * 16 + lane
                            dl = jnp.minimum(d16[lane] - h_lo, 160)
                            w = e16[lane]
                            qq = q16[lane]
                            for qs in range(H // 16):
                                sl = pl.ds(qs * 16, 16)
                                r = rows[b, e, sl]
                                sacc[dl, sl] = sacc[dl, sl] + r * w
                                gacc[dl, sl] = gacc[dl, sl] + r * qq
                                macc[dl, sl] = jnp.maximum(macc[dl, sl], r)
                        return 0
                    lax.fori_loop(0, SUBD // 16, grp, 0)
                    return 0
                lax.fori_loop(0, nsub_c, sub, 0)
                return 0
            lax.fori_loop(0, nch, chunk_body, 0)

            def fx(r, _):
                for q in range(H // 16):
                    sl = pl.ds(q * 16, 16)
                    v = macc[r, sl]
                    macc[r, sl] = jnp.where(v < -1.0e38, 0.0, v)
                return 0
            lax.fori_loop(0, HLOC, fx, 0)
            obase = pl.multiple_of(h_lo, 8)
            pltpu.sync_copy(sacc.at[pl.ds(0, HLOC)],
                            sum_out.at[pl.ds(obase, HLOC)])
            pltpu.sync_copy(gacc.at[pl.ds(0, HLOC)],
                            gcn_out.at[pl.ds(obase, HLOC)])
            pltpu.sync_copy(macc.at[pl.ds(0, HLOC)],
                            mx_out.at[pl.ds(obase, HLOC)])

    return kern


def _make_kernel_p():
    BP = 2000

    @functools.partial(
        pl.kernel,
        out_type=[jax.ShapeDtypeStruct((NW, 4, H), jnp.float32)],
        mesh=_sc_mesh(),
        compiler_params=_params_sc,
        scratch_types=[
            pltpu.VMEM((BP,), jnp.int32),
            pltpu.VMEM((BP + SUB,), jnp.int32),
            pltpu.VMEM((BP + SUB,), jnp.int32),
            pltpu.VMEM((8, H), jnp.float32),
            pltpu.VMEM((8, H), jnp.float32),
            pltpu.VMEM((SUB, H), jnp.float32),
            pltpu.VMEM((16,), jnp.float32),
            pltpu.VMEM((8, H), jnp.float32),
            pltpu.SMEM((8,), jnp.int32),
            pltpu.SemaphoreType.DMA,
        ],
    )
    def kern(merge_hbm, batch_hbm, pw_hbm, ro_out,
             bbuf, nid, gid, psum, pmax, rows, pwv, ost, cnt_s, sem):
        cid, sid, wid = _widx()
        glo = wid * 4
        iota = lax.iota(jnp.int32, 16)
        zf = jnp.zeros((16,), jnp.float32)
        negf = jnp.full((16,), NEG, jnp.float32)
        for r in range(8):
            for q in range(H // 16):
                psum[r, pl.ds(q * 16, 16)] = zf
                pmax[r, pl.ds(q * 16, 16)] = negf
            cnt_s[r] = 0
        pltpu.sync_copy(pw_hbm, pwv)

        for b5 in range(N // BP):
            pltpu.sync_copy(batch_hbm.at[pl.ds(b5 * BP, BP)], bbuf)

            def vb(v, cur):
                sl = pl.ds(v * 16, 16)
                b16 = bbuf[sl]
                m = (b16 >= glo) & (b16 < glo + 4)
                mi = m.astype(jnp.int32)
                pos = cur + plsc.cumsum(mi) - mi
                plsc.store_scatter(nid, [pos], b5 * BP + v * 16 + iota, mask=m)
                plsc.store_scatter(gid, [pos], b16 - glo, mask=m)
                return cur + jnp.max(plsc.all_reduce_population_count(m))
            cursor = lax.fori_loop(0, BP // 16, vb, jnp.int32(0))

            nsub = (cursor + SUB - 1) // SUB
            tbase = (cursor // 16) * 16
            for g in range(SUB // 16 + 1):
                idx = tbase + g * 16 + iota
                mm = (idx >= cursor) & (idx < nsub * SUB)
                plsc.store_scatter(nid, [idx], jnp.zeros((16,), jnp.int32),
                                   mask=mm)
                plsc.store_scatter(gid, [idx], jnp.full((16,), 4, jnp.int32),
                                   mask=mm)

            def sub_body(i, _):
                pltpu.async_copy(merge_hbm.at[nid.at[pl.ds(i * SUB, SUB)]],
                                 rows, sem).wait()

                def grp(g, _2):
                    g16 = gid[pl.ds(i * SUB + g * 16, 16)]
                    for lane in range(16):
                        e = g * 16 + lane
                        gl = jnp.minimum(g16[lane], 4)
                        cnt_s[gl] = cnt_s[gl] + 1
                        for qs in range(H // 16):
                            sl = pl.ds(qs * 16, 16)
                            r = rows[e, sl]
                            psum[gl, sl] = psum[gl, sl] + r
                            pmax[gl, sl] = jnp.maximum(pmax[gl, sl], r)
                    return 0
                lax.fori_loop(0, SUB // 16, grp, 0)
                return 0
            lax.fori_loop(0, nsub, sub_body, 0)

        pw = pwv[...]
        for gg in range(4):
            c = jnp.full((16,), cnt_s[gg], jnp.int32).astype(jnp.float32)
            coef = pw[0] * _recip16(jnp.maximum(c, 1.0)) + pw[1]
            for qs in range(H // 16):
                sl = pl.ds(qs * 16, 16)
                mx = pmax[gg, sl]
                mx = jnp.where(mx < -1.0e38, 0.0, mx)
                ost[gg, sl] = coef * psum[gg, sl] + pw[2] * mx
        pltpu.sync_copy(ost.at[pl.ds(0, 4)], ro_out.at[wid])

    return kern


_KS = None


def _get_kernels():
    global _KS
    if _KS is None:
        _KS = (_make_kernel_a(), _make_kernel_d(), _make_kernel_p())
    return _KS


def kernel(x, edge_weight, h_info_node, lin1_W, lin1_b, W_na,
           na_alphas, sc_alphas, la_alphas, pool_alphas, edge_index, batch):
    ka, kd, kp = _get_kernels()
    L = W_na.shape[0]
    na_w = jax.nn.softmax(na_alphas, axis=-1)
    sc_w = jax.nn.softmax(sc_alphas, axis=-1)
    la_w = jax.nn.softmax(la_alphas, axis=-1)
    pool_w = jax.nn.softmax(pool_alphas, axis=-1)
    src = edge_index[0]
    dst = edge_index[1]

    x_pad = jnp.pad(x, ((0, N2 - N), (0, 0)))
    b2d = jnp.broadcast_to(lin1_b[None, :], (8, H))

    lsrc, lew, ldst, cnts, isq, invc = ka(src, dst, edge_weight)
    h = _lin1(x_pad, lin1_W, b2d)
    invc2d = jnp.broadcast_to(invc[:, None], (N2, H))

    w0 = na_w[0:1]
    msc = jnp.concatenate([sc_w[0, 1:2], sc_w[1, 1:2], la_w[0],
                           jnp.zeros((3,), jnp.float32)])[None, :]

    hs = []
    for l in range(L):
        sum_agg, gcn_agg, max_agg = kd(h, lsrc, lew, ldst, cnts, isq)
        if l < L - 1:
            h = _layer(h, sum_agg, max_agg, gcn_agg, invc2d, W_na[l], w0)
            hs.append(h)
        else:
            merge = _layer3(h, sum_agg, max_agg, gcn_agg, invc2d, W_na[l],
                            w0, hs[0], hs[1], msc)

    pw = jnp.concatenate([pool_w[0], jnp.zeros((13,), jnp.float32)])
    ro = kp(merge, batch, pw)[0]
    return ro.reshape(G, H)
